# Initial kernel scaffold; baseline (speedup 1.0000x reference)
import jax
import jax.numpy as jnp
from jax.experimental import pallas as pl


def kernel(x, edge_index, W_conv, b_conv, W_lin, b_lin):
    raise NotImplementedError("write your pallas kernel here")


# speedup vs baseline: 10.5339x; 10.5339x over previous
import functools

import jax
import jax.numpy as jnp
from jax import lax
from jax.experimental import pallas as pl
from jax.experimental.pallas import tpu as pltpu
from jax.experimental.pallas import tpu_sc as plsc

N = 10000
E = 320000
D = 128

NC = 2
NS = 16
ROWS_PER_TILE = 624
ROWS_TAIL = N - NS * ROWS_PER_TILE
CHUNK = 80

DEG_EDGES_PER_WORKER = E // (NC * NS)
DEG_NCHUNK = DEG_EDGES_PER_WORKER // CHUNK
EDGES_PER_TILE = E // NS
NCHUNK = EDGES_PER_TILE // CHUNK


def _deg_kernel(dst):
    @functools.partial(
        pl.kernel,
        out_type=jax.ShapeDtypeStruct((NC * N,), jnp.float32),
        mesh=plsc.VectorSubcoreMesh(core_axis_name="c", subcore_axis_name="s"),
        scratch_types=[
            pltpu.VMEM_SHARED((N,), jnp.float32),
            pltpu.VMEM((ROWS_PER_TILE,), jnp.float32),
            pltpu.VMEM((CHUNK,), jnp.int32),
            pltpu.VMEM((CHUNK,), jnp.float32),
        ],
    )
    def k(dst_hbm, out_hbm, acc_sh, zeros_v, idx_v, ones_v):
        cid = lax.axis_index("c")
        sid = lax.axis_index("s")

        def fill_zeros(i, carry):
            zeros_v[pl.ds(i * 16, 16)] = jnp.zeros((16,), jnp.float32)
            return carry

        lax.fori_loop(0, ROWS_PER_TILE // 16, fill_zeros, 0)

        def fill_ones(i, carry):
            ones_v[pl.ds(i * 16, 16)] = jnp.ones((16,), jnp.float32)
            return carry

        lax.fori_loop(0, CHUNK // 16, fill_ones, 0)

        row0 = pl.multiple_of(sid * ROWS_PER_TILE, 8)
        pltpu.sync_copy(zeros_v, acc_sh.at[pl.ds(row0, ROWS_PER_TILE)])

        @pl.when(sid == NS - 1)
        def _():
            pltpu.sync_copy(zeros_v.at[pl.ds(0, ROWS_TAIL)],
                            acc_sh.at[pl.ds(NS * ROWS_PER_TILE, ROWS_TAIL)])

        plsc.subcore_barrier()

        base0 = (cid * NS + sid) * DEG_EDGES_PER_WORKER

        def body(i, carry):
            pltpu.sync_copy(dst_hbm.at[pl.ds(base0 + i * CHUNK, CHUNK)], idx_v)
            pltpu.sync_copy(ones_v, acc_sh.at[idx_v], add=True)
            return carry

        lax.fori_loop(0, DEG_NCHUNK, body, 0)
        plsc.subcore_barrier()
        obase = pl.multiple_of(cid * N + row0, 8)
        pltpu.sync_copy(acc_sh.at[pl.ds(row0, ROWS_PER_TILE)], zeros_v)
        pltpu.sync_copy(zeros_v, out_hbm.at[pl.ds(obase, ROWS_PER_TILE)])

        @pl.when(sid == NS - 1)
        def _():
            pltpu.sync_copy(acc_sh.at[pl.ds(NS * ROWS_PER_TILE, ROWS_TAIL)],
                            zeros_v.at[pl.ds(0, ROWS_TAIL)])
            pltpu.sync_copy(
                zeros_v.at[pl.ds(0, ROWS_TAIL)],
                out_hbm.at[pl.ds(pl.multiple_of(cid * N + NS * ROWS_PER_TILE, 8),
                                 ROWS_TAIL)])

    return k(dst)


HALF = N // 2
HALFP = 5120
ROWS_EDGE = HALFP // NS
NTRASH = 7


def _edge_kernel(y, src, dst):
    @functools.partial(
        pl.kernel,
        out_type=jax.ShapeDtypeStruct((N, D), jnp.float32),
        mesh=plsc.VectorSubcoreMesh(core_axis_name="c", subcore_axis_name="s"),
        scratch_types=[
            pltpu.VMEM_SHARED((HALFP, D), jnp.float32),
            pltpu.VMEM((ROWS_EDGE, D), jnp.float32),
            pltpu.VMEM((CHUNK,), jnp.int32),
            pltpu.VMEM((CHUNK,), jnp.int32),
            pltpu.VMEM((CHUNK,), jnp.int32),
            pltpu.VMEM((CHUNK, D), jnp.float32),
            pltpu.SemaphoreType.DMA,
        ],
    )
    def k(y_hbm, src_hbm, dst_hbm, out_hbm, acc_sh, zeros_v, sidx_v, didx_v,
          didx2_v, rows_v, sem):
        cid = lax.axis_index("c")
        sid = lax.axis_index("s")

        def fill_zeros(i, carry):
            for j in range(D // 16):
                zeros_v[i, pl.ds(j * 16, 16)] = jnp.zeros((16,), jnp.float32)
            return carry

        lax.fori_loop(0, ROWS_EDGE, fill_zeros, 0)
        row0 = pl.multiple_of(sid * ROWS_EDGE, 8)
        pltpu.sync_copy(zeros_v, acc_sh.at[pl.ds(row0, ROWS_EDGE)])
        plsc.subcore_barrier()

        base0 = sid * EDGES_PER_TILE
        lo = cid * HALF
        lane = lax.iota(jnp.int32, 16)

        def body(i, carry):
            base = base0 + i * CHUNK
            pltpu.sync_copy(src_hbm.at[pl.ds(base, CHUNK)], sidx_v)
            pltpu.sync_copy(dst_hbm.at[pl.ds(base, CHUNK)], didx_v)
            for j in range(CHUNK // 16):
                d = didx_v[pl.ds(j * 16, 16)]
                dloc = d - lo
                mine = (dloc >= 0) & (dloc < HALF)
                trash = HALF + lax.rem(i + j, NTRASH) * 16 + lane
                didx2_v[pl.ds(j * 16, 16)] = jnp.where(mine, dloc, trash)
            pltpu.async_copy(y_hbm.at[sidx_v], rows_v, sem).wait()
            pltpu.sync_copy(rows_v, acc_sh.at[didx2_v], add=True)
            return carry

        lax.fori_loop(0, NCHUNK, body, 0)
        plsc.subcore_barrier()

        @pl.when(sid < NS - 1)
        def _():
            crow = pl.multiple_of(sid * 312, 8)
            pltpu.sync_copy(acc_sh.at[pl.ds(crow, 312)],
                            zeros_v.at[pl.ds(0, 312)])
            pltpu.sync_copy(zeros_v.at[pl.ds(0, 312)],
                            out_hbm.at[pl.ds(pl.multiple_of(cid * HALF + crow, 8),
                                             312)])

        @pl.when(sid == NS - 1)
        def _():
            pltpu.sync_copy(acc_sh.at[pl.ds(4680, 320)], zeros_v)
            pltpu.sync_copy(zeros_v,
                            out_hbm.at[pl.ds(pl.multiple_of(cid * HALF + 4680, 8),
                                             320)])

    return k(y, src, dst)


_BN = 1000


def _prescale_body(x_ref, w_ref, d0_ref, d1_ref, y_ref):
    deg = d0_ref[...] + d1_ref[...] + 1.0
    dis = lax.rsqrt(deg)
    xw = jnp.dot(x_ref[...], w_ref[...], preferred_element_type=jnp.float32)
    y_ref[...] = xw * dis


def _prescale(x, w, deg0, deg1):
    return pl.pallas_call(
        _prescale_body,
        grid=(N // _BN,),
        in_specs=[
            pl.BlockSpec((_BN, D), lambda i: (i, 0)),
            pl.BlockSpec((D, D), lambda i: (0, 0)),
            pl.BlockSpec((_BN, 1), lambda i: (i, 0)),
            pl.BlockSpec((_BN, 1), lambda i: (i, 0)),
        ],
        out_specs=pl.BlockSpec((_BN, D), lambda i: (i, 0)),
        out_shape=jax.ShapeDtypeStruct((N, D), jnp.float32),
    )(x, w, deg0, deg1)


def _finish_body(a_ref, y_ref, d0_ref, d1_ref, bc_ref, wl_ref, bl_ref,
                 h_ref, o_ref):
    deg = d0_ref[...] + d1_ref[...] + 1.0
    dis = lax.rsqrt(deg)
    s = (a_ref[...] + y_ref[...]) * dis + bc_ref[...]
    h = jnp.maximum(s, 0.0)
    h_ref[...] = h
    z = jnp.dot(h, wl_ref[...], preferred_element_type=jnp.float32) + bl_ref[...]
    o_ref[...] = jax.nn.sigmoid(z)


def _finish(acc, y, deg0, deg1, bc, wl, bl):
    return pl.pallas_call(
        _finish_body,
        grid=(N // _BN,),
        in_specs=[
            pl.BlockSpec((_BN, D), lambda i: (i, 0)),
            pl.BlockSpec((_BN, D), lambda i: (i, 0)),
            pl.BlockSpec((_BN, 1), lambda i: (i, 0)),
            pl.BlockSpec((_BN, 1), lambda i: (i, 0)),
            pl.BlockSpec((1, D), lambda i: (0, 0)),
            pl.BlockSpec((D, 1), lambda i: (0, 0)),
            pl.BlockSpec((1, 1), lambda i: (0, 0)),
        ],
        out_specs=[
            pl.BlockSpec((_BN, D), lambda i: (i, 0)),
            pl.BlockSpec((_BN, 1), lambda i: (i, 0)),
        ],
        out_shape=[
            jax.ShapeDtypeStruct((N, D), jnp.float32),
            jax.ShapeDtypeStruct((N, 1), jnp.float32),
        ],
    )(acc, y, deg0, deg1, bc, wl, bl)


def kernel(x, edge_index, W_conv, b_conv, W_lin, b_lin):
    src = edge_index[0].astype(jnp.int32)
    dst = edge_index[1].astype(jnp.int32)

    degw = _deg_kernel(dst)
    deg0 = degw[:N].reshape(N, 1)
    deg1 = degw[N:].reshape(N, 1)

    y = _prescale(x, W_conv, deg0, deg1)
    acc = _edge_kernel(y, src, dst)

    h, out = _finish(acc, y, deg0, deg1,
                     b_conv.reshape(1, D), W_lin, b_lin.reshape(1, 1))
    return (out, h)


# speedup vs baseline: 22.0551x; 2.0937x over previous
import functools

import jax
import jax.numpy as jnp
from jax import lax
from jax.experimental import pallas as pl
from jax.experimental.pallas import tpu as pltpu
from jax.experimental.pallas import tpu_sc as plsc

N = 10000
E = 320000
D = 128

NC = 2
NS = 16
ROWS_PER_TILE = 624
ROWS_TAIL = N - NS * ROWS_PER_TILE
CHUNK = 80

DEG_EDGES_PER_WORKER = E // (NC * NS)
DEG_NCHUNK = DEG_EDGES_PER_WORKER // CHUNK
EDGES_PER_TILE = E // NS
NCHUNK = EDGES_PER_TILE // CHUNK


def _deg_kernel(dst):
    @functools.partial(
        pl.kernel,
        out_type=jax.ShapeDtypeStruct((NC * N,), jnp.float32),
        mesh=plsc.VectorSubcoreMesh(core_axis_name="c", subcore_axis_name="s"),
        scratch_types=[
            pltpu.VMEM_SHARED((N,), jnp.float32),
            pltpu.VMEM((ROWS_PER_TILE,), jnp.float32),
            pltpu.VMEM((2, CHUNK), jnp.int32),
            pltpu.VMEM((CHUNK,), jnp.float32),
            pltpu.SemaphoreType.DMA((2,)),
        ],
    )
    def k(dst_hbm, out_hbm, acc_sh, zeros_v, idx_v, ones_v, isem):
        cid = lax.axis_index("c")
        sid = lax.axis_index("s")

        def fill_zeros(i, carry):
            zeros_v[pl.ds(i * 16, 16)] = jnp.zeros((16,), jnp.float32)
            return carry

        lax.fori_loop(0, ROWS_PER_TILE // 16, fill_zeros, 0)

        def fill_ones(i, carry):
            ones_v[pl.ds(i * 16, 16)] = jnp.ones((16,), jnp.float32)
            return carry

        lax.fori_loop(0, CHUNK // 16, fill_ones, 0)

        row0 = pl.multiple_of(sid * ROWS_PER_TILE, 8)
        pltpu.sync_copy(zeros_v, acc_sh.at[pl.ds(row0, ROWS_PER_TILE)])

        @pl.when(sid == NS - 1)
        def _():
            pltpu.sync_copy(zeros_v.at[pl.ds(0, ROWS_TAIL)],
                            acc_sh.at[pl.ds(NS * ROWS_PER_TILE, ROWS_TAIL)])

        plsc.subcore_barrier()

        base0 = (cid * NS + sid) * DEG_EDGES_PER_WORKER

        def issue(i, b):
            pltpu.async_copy(dst_hbm.at[pl.ds(base0 + i * CHUNK, CHUNK)],
                             idx_v.at[b], isem.at[b])

        def wait(i, b):
            pltpu.make_async_copy(dst_hbm.at[pl.ds(base0 + i * CHUNK, CHUNK)],
                                  idx_v.at[b], isem.at[b]).wait()

        def scatter(b):
            pltpu.sync_copy(ones_v, acc_sh.at[idx_v.at[b]], add=True)

        issue(0, 0)

        def body(t, carry):
            i = 2 * t
            wait(i, 0)
            issue(i + 1, 1)
            scatter(0)
            wait(i + 1, 1)
            issue(i + 2, 0)
            scatter(1)
            return carry

        lax.fori_loop(0, (DEG_NCHUNK - 1) // 2, body, 0)
        wait(DEG_NCHUNK - 1, 0)
        scatter(0)
        plsc.subcore_barrier()
        obase = pl.multiple_of(cid * N + row0, 8)
        pltpu.sync_copy(acc_sh.at[pl.ds(row0, ROWS_PER_TILE)], zeros_v)
        pltpu.sync_copy(zeros_v, out_hbm.at[pl.ds(obase, ROWS_PER_TILE)])

        @pl.when(sid == NS - 1)
        def _():
            pltpu.sync_copy(acc_sh.at[pl.ds(NS * ROWS_PER_TILE, ROWS_TAIL)],
                            zeros_v.at[pl.ds(0, ROWS_TAIL)])
            pltpu.sync_copy(
                zeros_v.at[pl.ds(0, ROWS_TAIL)],
                out_hbm.at[pl.ds(pl.multiple_of(cid * N + NS * ROWS_PER_TILE, 8),
                                 ROWS_TAIL)])

    return k(dst)


HALF = N // 2
HALFP = 5120
ROWS_EDGE = HALFP // NS
NTRASH = 7


def _edge_kernel(y, src, dst):
    @functools.partial(
        pl.kernel,
        out_type=jax.ShapeDtypeStruct((N, D), jnp.float32),
        mesh=plsc.VectorSubcoreMesh(core_axis_name="c", subcore_axis_name="s"),
        scratch_types=[
            pltpu.VMEM_SHARED((HALFP, D), jnp.float32),
            pltpu.VMEM((ROWS_EDGE, D), jnp.float32),
            pltpu.VMEM((2, CHUNK), jnp.int32),
            pltpu.VMEM((2, CHUNK), jnp.int32),
            pltpu.VMEM((2, CHUNK), jnp.int32),
            pltpu.VMEM((2, CHUNK, D), jnp.float32),
            pltpu.SemaphoreType.DMA((2,)),
            pltpu.SemaphoreType.DMA((2,)),
        ],
    )
    def k(y_hbm, src_hbm, dst_hbm, out_hbm, acc_sh, zeros_v, sidx_v, didx_v,
          didx2_v, rows_v, isem, gsem):
        cid = lax.axis_index("c")
        sid = lax.axis_index("s")

        def fill_zeros(i, carry):
            for j in range(D // 16):
                zeros_v[i, pl.ds(j * 16, 16)] = jnp.zeros((16,), jnp.float32)
            return carry

        lax.fori_loop(0, ROWS_EDGE, fill_zeros, 0)
        row0 = pl.multiple_of(sid * ROWS_EDGE, 8)
        pltpu.sync_copy(zeros_v, acc_sh.at[pl.ds(row0, ROWS_EDGE)])
        plsc.subcore_barrier()

        base0 = sid * EDGES_PER_TILE
        lo = cid * HALF
        lane = lax.iota(jnp.int32, 16)

        def idx_issue(i, b):
            base = base0 + i * CHUNK
            pltpu.async_copy(src_hbm.at[pl.ds(base, CHUNK)], sidx_v.at[b],
                             isem.at[b])
            pltpu.async_copy(dst_hbm.at[pl.ds(base, CHUNK)], didx_v.at[b],
                             isem.at[b])

        def idx_wait(i, b):
            base = base0 + i * CHUNK
            pltpu.make_async_copy(src_hbm.at[pl.ds(base, CHUNK)], sidx_v.at[b],
                                  isem.at[b]).wait()
            pltpu.make_async_copy(dst_hbm.at[pl.ds(base, CHUNK)], didx_v.at[b],
                                  isem.at[b]).wait()

        def remap(i, b):
            for j in range(CHUNK // 16):
                d = didx_v[b, pl.ds(j * 16, 16)]
                dloc = d - lo
                mine = (dloc >= 0) & (dloc < HALF)
                trash = HALF + lax.rem(i + j, NTRASH) * 16 + lane
                didx2_v[b, pl.ds(j * 16, 16)] = jnp.where(mine, dloc, trash)

        def gather_issue(b):
            pltpu.async_copy(y_hbm.at[sidx_v.at[b]], rows_v.at[b], gsem.at[b])

        def gather_wait(b):
            pltpu.make_async_copy(y_hbm.at[sidx_v.at[b]], rows_v.at[b],
                                  gsem.at[b]).wait()

        def scatter(b):
            pltpu.sync_copy(rows_v.at[b], acc_sh.at[didx2_v.at[b]], add=True)

        idx_issue(0, 0)
        idx_issue(1, 1)
        idx_wait(0, 0)
        remap(0, 0)
        gather_issue(0)

        def body(t, carry):
            i0 = 2 * t
            idx_wait(i0 + 1, 1)
            remap(i0 + 1, 1)
            gather_issue(1)
            gather_wait(0)
            idx_issue(i0 + 2, 0)
            scatter(0)
            idx_wait(i0 + 2, 0)
            remap(i0 + 2, 0)
            gather_issue(0)
            gather_wait(1)
            idx_issue(i0 + 3, 1)
            scatter(1)
            return carry

        lax.fori_loop(0, NCHUNK // 2 - 1, body, 0)
        idx_wait(NCHUNK - 1, 1)
        remap(NCHUNK - 1, 1)
        gather_issue(1)
        gather_wait(0)
        scatter(0)
        gather_wait(1)
        scatter(1)
        plsc.subcore_barrier()

        @pl.when(sid < NS - 1)
        def _():
            crow = pl.multiple_of(sid * 312, 8)
            pltpu.sync_copy(acc_sh.at[pl.ds(crow, 312)],
                            zeros_v.at[pl.ds(0, 312)])
            pltpu.sync_copy(zeros_v.at[pl.ds(0, 312)],
                            out_hbm.at[pl.ds(pl.multiple_of(cid * HALF + crow, 8),
                                             312)])

        @pl.when(sid == NS - 1)
        def _():
            pltpu.sync_copy(acc_sh.at[pl.ds(4680, 320)], zeros_v)
            pltpu.sync_copy(zeros_v,
                            out_hbm.at[pl.ds(pl.multiple_of(cid * HALF + 4680, 8),
                                             320)])

    return k(y, src, dst)


_BN = 1000


def _prescale_body(x_ref, w_ref, d0_ref, d1_ref, y_ref):
    deg = d0_ref[...] + d1_ref[...] + 1.0
    dis = lax.rsqrt(deg)
    xw = jnp.dot(x_ref[...], w_ref[...], preferred_element_type=jnp.float32)
    y_ref[...] = xw * dis


def _prescale(x, w, deg0, deg1):
    return pl.pallas_call(
        _prescale_body,
        grid=(N // _BN,),
        in_specs=[
            pl.BlockSpec((_BN, D), lambda i: (i, 0)),
            pl.BlockSpec((D, D), lambda i: (0, 0)),
            pl.BlockSpec((_BN, 1), lambda i: (i, 0)),
            pl.BlockSpec((_BN, 1), lambda i: (i, 0)),
        ],
        out_specs=pl.BlockSpec((_BN, D), lambda i: (i, 0)),
        out_shape=jax.ShapeDtypeStruct((N, D), jnp.float32),
    )(x, w, deg0, deg1)


def _finish_body(a_ref, y_ref, d0_ref, d1_ref, bc_ref, wl_ref, bl_ref,
                 h_ref, o_ref):
    deg = d0_ref[...] + d1_ref[...] + 1.0
    dis = lax.rsqrt(deg)
    s = (a_ref[...] + y_ref[...]) * dis + bc_ref[...]
    h = jnp.maximum(s, 0.0)
    h_ref[...] = h
    z = jnp.dot(h, wl_ref[...], preferred_element_type=jnp.float32) + bl_ref[...]
    o_ref[...] = jax.nn.sigmoid(z)


def _finish(acc, y, deg0, deg1, bc, wl, bl):
    return pl.pallas_call(
        _finish_body,
        grid=(N // _BN,),
        in_specs=[
            pl.BlockSpec((_BN, D), lambda i: (i, 0)),
            pl.BlockSpec((_BN, D), lambda i: (i, 0)),
            pl.BlockSpec((_BN, 1), lambda i: (i, 0)),
            pl.BlockSpec((_BN, 1), lambda i: (i, 0)),
            pl.BlockSpec((1, D), lambda i: (0, 0)),
            pl.BlockSpec((D, 1), lambda i: (0, 0)),
            pl.BlockSpec((1, 1), lambda i: (0, 0)),
        ],
        out_specs=[
            pl.BlockSpec((_BN, D), lambda i: (i, 0)),
            pl.BlockSpec((_BN, 1), lambda i: (i, 0)),
        ],
        out_shape=[
            jax.ShapeDtypeStruct((N, D), jnp.float32),
            jax.ShapeDtypeStruct((N, 1), jnp.float32),
        ],
    )(acc, y, deg0, deg1, bc, wl, bl)


def kernel(x, edge_index, W_conv, b_conv, W_lin, b_lin):
    src = edge_index[0].astype(jnp.int32)
    dst = edge_index[1].astype(jnp.int32)

    degw = _deg_kernel(dst)
    deg0 = degw[:N].reshape(N, 1)
    deg1 = degw[N:].reshape(N, 1)

    y = _prescale(x, W_conv, deg0, deg1)
    acc = _edge_kernel(y, src, dst)

    h, out = _finish(acc, y, deg0, deg1,
                     b_conv.reshape(1, D), W_lin, b_lin.reshape(1, 1))
    return (out, h)


# speedup vs baseline: 25.7940x; 1.1695x over previous
import functools

import jax
import jax.numpy as jnp
from jax import lax
from jax.experimental import pallas as pl
from jax.experimental.pallas import tpu as pltpu
from jax.experimental.pallas import tpu_sc as plsc

N = 10000
E = 320000
D = 128

NC = 2
NS = 16
ROWS_PER_TILE = 624
ROWS_TAIL = N - NS * ROWS_PER_TILE
CHUNK = 128

DEG_EDGES_PER_WORKER = E // (NC * NS)
DEG_NFULL = DEG_EDGES_PER_WORKER // CHUNK
DEG_TAIL = DEG_EDGES_PER_WORKER - DEG_NFULL * CHUNK
EDGES_PER_TILE = E // NS
NFULL = EDGES_PER_TILE // CHUNK
TAIL = EDGES_PER_TILE - NFULL * CHUNK


def _deg_kernel(dst):
    @functools.partial(
        pl.kernel,
        out_type=jax.ShapeDtypeStruct((NC * N,), jnp.float32),
        mesh=plsc.VectorSubcoreMesh(core_axis_name="c", subcore_axis_name="s"),
        scratch_types=[
            pltpu.VMEM_SHARED((N,), jnp.float32),
            pltpu.VMEM((ROWS_PER_TILE,), jnp.float32),
            pltpu.VMEM((2, CHUNK), jnp.int32),
            pltpu.VMEM((CHUNK,), jnp.float32),
            pltpu.VMEM((DEG_TAIL,), jnp.int32),
            pltpu.SemaphoreType.DMA((2,)),
        ],
    )
    def k(dst_hbm, out_hbm, acc_sh, zeros_v, idx_v, ones_v, tidx_v, isem):
        cid = lax.axis_index("c")
        sid = lax.axis_index("s")

        def fill_zeros(i, carry):
            zeros_v[pl.ds(i * 16, 16)] = jnp.zeros((16,), jnp.float32)
            return carry

        lax.fori_loop(0, ROWS_PER_TILE // 16, fill_zeros, 0)

        def fill_ones(i, carry):
            ones_v[pl.ds(i * 16, 16)] = jnp.ones((16,), jnp.float32)
            return carry

        lax.fori_loop(0, CHUNK // 16, fill_ones, 0)

        row0 = pl.multiple_of(sid * ROWS_PER_TILE, 8)
        pltpu.sync_copy(zeros_v, acc_sh.at[pl.ds(row0, ROWS_PER_TILE)])

        @pl.when(sid == NS - 1)
        def _():
            pltpu.sync_copy(zeros_v.at[pl.ds(0, ROWS_TAIL)],
                            acc_sh.at[pl.ds(NS * ROWS_PER_TILE, ROWS_TAIL)])

        plsc.subcore_barrier()

        base0 = (cid * NS + sid) * DEG_EDGES_PER_WORKER

        def issue(i, b):
            pltpu.async_copy(dst_hbm.at[pl.ds(base0 + i * CHUNK, CHUNK)],
                             idx_v.at[b], isem.at[b])

        def wait(i, b):
            pltpu.make_async_copy(dst_hbm.at[pl.ds(base0 + i * CHUNK, CHUNK)],
                                  idx_v.at[b], isem.at[b]).wait()

        def scatter(b):
            pltpu.sync_copy(ones_v, acc_sh.at[idx_v.at[b]], add=True)

        tslice = dst_hbm.at[pl.ds(base0 + DEG_NFULL * CHUNK, DEG_TAIL)]

        issue(0, 0)

        def body(t, carry):
            i = 2 * t
            wait(i, 0)
            issue(i + 1, 1)
            scatter(0)
            wait(i + 1, 1)
            issue(i + 2, 0)
            scatter(1)
            return carry

        lax.fori_loop(0, DEG_NFULL // 2 - 1, body, 0)
        wait(DEG_NFULL - 2, 0)
        issue(DEG_NFULL - 1, 1)
        pltpu.async_copy(tslice, tidx_v, isem.at[0])
        scatter(0)
        wait(DEG_NFULL - 1, 1)
        scatter(1)
        pltpu.make_async_copy(tslice, tidx_v, isem.at[0]).wait()
        pltpu.sync_copy(ones_v.at[pl.ds(0, DEG_TAIL)], acc_sh.at[tidx_v],
                        add=True)
        plsc.subcore_barrier()
        obase = pl.multiple_of(cid * N + row0, 8)
        pltpu.sync_copy(acc_sh.at[pl.ds(row0, ROWS_PER_TILE)], zeros_v)
        pltpu.sync_copy(zeros_v, out_hbm.at[pl.ds(obase, ROWS_PER_TILE)])

        @pl.when(sid == NS - 1)
        def _():
            pltpu.sync_copy(acc_sh.at[pl.ds(NS * ROWS_PER_TILE, ROWS_TAIL)],
                            zeros_v.at[pl.ds(0, ROWS_TAIL)])
            pltpu.sync_copy(
                zeros_v.at[pl.ds(0, ROWS_TAIL)],
                out_hbm.at[pl.ds(pl.multiple_of(cid * N + NS * ROWS_PER_TILE, 8),
                                 ROWS_TAIL)])

    return k(dst)


HALF = N // 2
HALFP = 5120
ROWS_EDGE = HALFP // NS
NTRASH = 7


def _edge_kernel(y, src, dst):
    @functools.partial(
        pl.kernel,
        out_type=jax.ShapeDtypeStruct((N, D), jnp.float32),
        mesh=plsc.VectorSubcoreMesh(core_axis_name="c", subcore_axis_name="s"),
        scratch_types=[
            pltpu.VMEM_SHARED((HALFP, D), jnp.float32),
            pltpu.VMEM((ROWS_EDGE, D), jnp.float32),
            pltpu.VMEM((2, CHUNK), jnp.int32),
            pltpu.VMEM((2, CHUNK), jnp.int32),
            pltpu.VMEM((2, CHUNK), jnp.int32),
            pltpu.VMEM((2, CHUNK, D), jnp.float32),
            pltpu.VMEM((TAIL,), jnp.int32),
            pltpu.VMEM((TAIL,), jnp.int32),
            pltpu.VMEM((TAIL,), jnp.int32),
            pltpu.VMEM((TAIL, D), jnp.float32),
            pltpu.SemaphoreType.DMA((2,)),
            pltpu.SemaphoreType.DMA((2,)),
        ],
    )
    def k(y_hbm, src_hbm, dst_hbm, out_hbm, acc_sh, zeros_v, sidx_v, didx_v,
          didx2_v, rows_v, tsidx_v, tdidx_v, tdidx2_v, trows_v, isem, gsem):
        cid = lax.axis_index("c")
        sid = lax.axis_index("s")

        def fill_zeros(i, carry):
            for j in range(D // 16):
                zeros_v[i, pl.ds(j * 16, 16)] = jnp.zeros((16,), jnp.float32)
            return carry

        lax.fori_loop(0, ROWS_EDGE, fill_zeros, 0)
        row0 = pl.multiple_of(sid * ROWS_EDGE, 8)
        pltpu.sync_copy(zeros_v, acc_sh.at[pl.ds(row0, ROWS_EDGE)])
        plsc.subcore_barrier()

        base0 = sid * EDGES_PER_TILE
        lo = cid * HALF
        lane = lax.iota(jnp.int32, 16)

        def idx_issue(i, b):
            base = base0 + i * CHUNK
            pltpu.async_copy(src_hbm.at[pl.ds(base, CHUNK)], sidx_v.at[b],
                             isem.at[b])
            pltpu.async_copy(dst_hbm.at[pl.ds(base, CHUNK)], didx_v.at[b],
                             isem.at[b])

        def idx_wait(i, b):
            base = base0 + i * CHUNK
            pltpu.make_async_copy(src_hbm.at[pl.ds(base, CHUNK)], sidx_v.at[b],
                                  isem.at[b]).wait()
            pltpu.make_async_copy(dst_hbm.at[pl.ds(base, CHUNK)], didx_v.at[b],
                                  isem.at[b]).wait()

        def remap(i, b):
            for j in range(CHUNK // 16):
                d = didx_v[b, pl.ds(j * 16, 16)]
                dloc = d - lo
                mine = (dloc >= 0) & (dloc < HALF)
                trash = HALF + lax.rem(i + j, NTRASH) * 16 + lane
                didx2_v[b, pl.ds(j * 16, 16)] = jnp.where(mine, dloc, trash)

        def gather_issue(b):
            pltpu.async_copy(y_hbm.at[sidx_v.at[b]], rows_v.at[b], gsem.at[b])

        def gather_wait(b):
            pltpu.make_async_copy(y_hbm.at[sidx_v.at[b]], rows_v.at[b],
                                  gsem.at[b]).wait()

        def scatter(b):
            pltpu.sync_copy(rows_v.at[b], acc_sh.at[didx2_v.at[b]], add=True)

        tbase = base0 + NFULL * CHUNK
        tsrc_slice = src_hbm.at[pl.ds(tbase, TAIL)]
        tdst_slice = dst_hbm.at[pl.ds(tbase, TAIL)]

        idx_issue(0, 0)
        idx_issue(1, 1)
        idx_wait(0, 0)
        remap(0, 0)
        gather_issue(0)

        def body(t, carry):
            i0 = 2 * t
            idx_wait(i0 + 1, 1)
            remap(i0 + 1, 1)
            gather_issue(1)
            gather_wait(0)
            idx_issue(i0 + 2, 0)
            scatter(0)
            idx_wait(i0 + 2, 0)
            remap(i0 + 2, 0)
            gather_issue(0)
            gather_wait(1)
            idx_issue(i0 + 3, 1)
            scatter(1)
            return carry

        lax.fori_loop(0, NFULL // 2 - 1, body, 0)
        idx_wait(NFULL - 1, 1)
        remap(NFULL - 1, 1)
        gather_issue(1)
        gather_wait(0)
        pltpu.async_copy(tsrc_slice, tsidx_v, isem.at[0])
        pltpu.async_copy(tdst_slice, tdidx_v, isem.at[0])
        scatter(0)
        pltpu.make_async_copy(tsrc_slice, tsidx_v, isem.at[0]).wait()
        pltpu.make_async_copy(tdst_slice, tdidx_v, isem.at[0]).wait()
        for j in range(TAIL // 16):
            d = tdidx_v[pl.ds(j * 16, 16)]
            dloc = d - lo
            mine = (dloc >= 0) & (dloc < HALF)
            trash = HALF + j * 16 + lane
            tdidx2_v[pl.ds(j * 16, 16)] = jnp.where(mine, dloc, trash)
        pltpu.async_copy(y_hbm.at[tsidx_v], trows_v, gsem.at[0])
        gather_wait(1)
        scatter(1)
        pltpu.make_async_copy(y_hbm.at[tsidx_v], trows_v, gsem.at[0]).wait()
        pltpu.sync_copy(trows_v, acc_sh.at[tdidx2_v], add=True)
        plsc.subcore_barrier()

        @pl.when(sid < NS - 1)
        def _():
            crow = pl.multiple_of(sid * 312, 8)
            pltpu.sync_copy(acc_sh.at[pl.ds(crow, 312)],
                            zeros_v.at[pl.ds(0, 312)])
            pltpu.sync_copy(zeros_v.at[pl.ds(0, 312)],
                            out_hbm.at[pl.ds(pl.multiple_of(cid * HALF + crow, 8),
                                             312)])

        @pl.when(sid == NS - 1)
        def _():
            pltpu.sync_copy(acc_sh.at[pl.ds(4680, 320)], zeros_v)
            pltpu.sync_copy(zeros_v,
                            out_hbm.at[pl.ds(pl.multiple_of(cid * HALF + 4680, 8),
                                             320)])

    return k(y, src, dst)


_BN = 1000


def _prescale_body(x_ref, w_ref, d0_ref, d1_ref, y_ref):
    deg = d0_ref[...] + d1_ref[...] + 1.0
    dis = lax.rsqrt(deg)
    xw = jnp.dot(x_ref[...], w_ref[...], preferred_element_type=jnp.float32)
    y_ref[...] = xw * dis


def _prescale(x, w, deg0, deg1):
    return pl.pallas_call(
        _prescale_body,
        grid=(N // _BN,),
        in_specs=[
            pl.BlockSpec((_BN, D), lambda i: (i, 0)),
            pl.BlockSpec((D, D), lambda i: (0, 0)),
            pl.BlockSpec((_BN, 1), lambda i: (i, 0)),
            pl.BlockSpec((_BN, 1), lambda i: (i, 0)),
        ],
        out_specs=pl.BlockSpec((_BN, D), lambda i: (i, 0)),
        out_shape=jax.ShapeDtypeStruct((N, D), jnp.float32),
    )(x, w, deg0, deg1)


def _finish_body(a_ref, y_ref, d0_ref, d1_ref, bc_ref, wl_ref, bl_ref,
                 h_ref, o_ref):
    deg = d0_ref[...] + d1_ref[...] + 1.0
    dis = lax.rsqrt(deg)
    s = (a_ref[...] + y_ref[...]) * dis + bc_ref[...]
    h = jnp.maximum(s, 0.0)
    h_ref[...] = h
    z = jnp.dot(h, wl_ref[...], preferred_element_type=jnp.float32) + bl_ref[...]
    o_ref[...] = jax.nn.sigmoid(z)


def _finish(acc, y, deg0, deg1, bc, wl, bl):
    return pl.pallas_call(
        _finish_body,
        grid=(N // _BN,),
        in_specs=[
            pl.BlockSpec((_BN, D), lambda i: (i, 0)),
            pl.BlockSpec((_BN, D), lambda i: (i, 0)),
            pl.BlockSpec((_BN, 1), lambda i: (i, 0)),
            pl.BlockSpec((_BN, 1), lambda i: (i, 0)),
            pl.BlockSpec((1, D), lambda i: (0, 0)),
            pl.BlockSpec((D, 1), lambda i: (0, 0)),
            pl.BlockSpec((1, 1), lambda i: (0, 0)),
        ],
        out_specs=[
            pl.BlockSpec((_BN, D), lambda i: (i, 0)),
            pl.BlockSpec((_BN, 1), lambda i: (i, 0)),
        ],
        out_shape=[
            jax.ShapeDtypeStruct((N, D), jnp.float32),
            jax.ShapeDtypeStruct((N, 1), jnp.float32),
        ],
    )(acc, y, deg0, deg1, bc, wl, bl)


def kernel(x, edge_index, W_conv, b_conv, W_lin, b_lin):
    src = edge_index[0].astype(jnp.int32)
    dst = edge_index[1].astype(jnp.int32)

    degw = _deg_kernel(dst)
    deg0 = degw[:N].reshape(N, 1)
    deg1 = degw[N:].reshape(N, 1)

    y = _prescale(x, W_conv, deg0, deg1)
    acc = _edge_kernel(y, src, dst)

    h, out = _finish(acc, y, deg0, deg1,
                     b_conv.reshape(1, D), W_lin, b_lin.reshape(1, 1))
    return (out, h)


# speedup vs baseline: 39.7189x; 1.5399x over previous
import functools

import jax
import jax.numpy as jnp
from jax import lax
from jax.experimental import pallas as pl
from jax.experimental.pallas import tpu as pltpu
from jax.experimental.pallas import tpu_sc as plsc

N = 10000
E = 320000
D = 128

NC = 2
NS = 16
ROWS_PER_TILE = 624
ROWS_TAIL = N - NS * ROWS_PER_TILE
CHUNK = 128

DEG_EDGES_PER_WORKER = E // (NC * NS)
DEG_NFULL = DEG_EDGES_PER_WORKER // CHUNK
DEG_TAIL = DEG_EDGES_PER_WORKER - DEG_NFULL * CHUNK


def _deg_kernel(dst):
    @functools.partial(
        pl.kernel,
        out_type=jax.ShapeDtypeStruct((NC * N,), jnp.float32),
        mesh=plsc.VectorSubcoreMesh(core_axis_name="c", subcore_axis_name="s"),
        scratch_types=[
            pltpu.VMEM_SHARED((N,), jnp.float32),
            pltpu.VMEM((ROWS_PER_TILE,), jnp.float32),
            pltpu.VMEM((3, CHUNK), jnp.int32),
            pltpu.VMEM((CHUNK,), jnp.float32),
            pltpu.VMEM((DEG_TAIL,), jnp.int32),
            pltpu.SemaphoreType.DMA((3,)),
        ],
    )
    def k(dst_hbm, out_hbm, acc_sh, zeros_v, idx_v, ones_v, tidx_v, isem):
        cid = lax.axis_index("c")
        sid = lax.axis_index("s")

        def fill_zeros(i, carry):
            zeros_v[pl.ds(i * 16, 16)] = jnp.zeros((16,), jnp.float32)
            return carry

        lax.fori_loop(0, ROWS_PER_TILE // 16, fill_zeros, 0)

        def fill_ones(i, carry):
            ones_v[pl.ds(i * 16, 16)] = jnp.ones((16,), jnp.float32)
            return carry

        lax.fori_loop(0, CHUNK // 16, fill_ones, 0)

        row0 = pl.multiple_of(sid * ROWS_PER_TILE, 8)
        pltpu.sync_copy(zeros_v, acc_sh.at[pl.ds(row0, ROWS_PER_TILE)])

        @pl.when(sid == NS - 1)
        def _():
            pltpu.sync_copy(zeros_v.at[pl.ds(0, ROWS_TAIL)],
                            acc_sh.at[pl.ds(NS * ROWS_PER_TILE, ROWS_TAIL)])

        plsc.subcore_barrier()

        base0 = (cid * NS + sid) * DEG_EDGES_PER_WORKER

        def issue(i, b):
            pltpu.async_copy(dst_hbm.at[pl.ds(base0 + i * CHUNK, CHUNK)],
                             idx_v.at[b], isem.at[b])

        def wait(i, b):
            pltpu.make_async_copy(dst_hbm.at[pl.ds(base0 + i * CHUNK, CHUNK)],
                                  idx_v.at[b], isem.at[b]).wait()

        def scatter(b):
            pltpu.sync_copy(ones_v, acc_sh.at[idx_v.at[b]], add=True)

        tslice = dst_hbm.at[pl.ds(base0 + DEG_NFULL * CHUNK, DEG_TAIL)]

        issue(0, 0)
        issue(1, 1)
        issue(2, 2)

        def body(t, carry):
            i = 3 * t
            for b in range(3):
                wait(i + b, b)
                scatter(b)
                issue(i + b + 3, b)
            return carry

        lax.fori_loop(0, DEG_NFULL // 3 - 1, body, 0)
        wait(DEG_NFULL - 3, 0)
        scatter(0)
        pltpu.async_copy(tslice, tidx_v, isem.at[0])
        wait(DEG_NFULL - 2, 1)
        scatter(1)
        wait(DEG_NFULL - 1, 2)
        scatter(2)
        pltpu.make_async_copy(tslice, tidx_v, isem.at[0]).wait()
        pltpu.sync_copy(ones_v.at[pl.ds(0, DEG_TAIL)], acc_sh.at[tidx_v],
                        add=True)
        plsc.subcore_barrier()
        obase = pl.multiple_of(cid * N + row0, 8)
        pltpu.sync_copy(acc_sh.at[pl.ds(row0, ROWS_PER_TILE)], zeros_v)
        pltpu.sync_copy(zeros_v, out_hbm.at[pl.ds(obase, ROWS_PER_TILE)])

        @pl.when(sid == NS - 1)
        def _():
            pltpu.sync_copy(acc_sh.at[pl.ds(NS * ROWS_PER_TILE, ROWS_TAIL)],
                            zeros_v.at[pl.ds(0, ROWS_TAIL)])
            pltpu.sync_copy(
                zeros_v.at[pl.ds(0, ROWS_TAIL)],
                out_hbm.at[pl.ds(pl.multiple_of(cid * N + NS * ROWS_PER_TILE, 8),
                                 ROWS_TAIL)])

    return k(dst)


E_NFULL = DEG_EDGES_PER_WORKER // CHUNK
E_TAIL = DEG_EDGES_PER_WORKER - E_NFULL * CHUNK


def _edge_kernel(y, src, dst):
    @functools.partial(
        pl.kernel,
        out_type=jax.ShapeDtypeStruct((NC * N, D), jnp.float32),
        mesh=plsc.VectorSubcoreMesh(core_axis_name="c", subcore_axis_name="s"),
        scratch_types=[
            pltpu.VMEM_SHARED((N, D), jnp.float32),
            pltpu.VMEM((2, CHUNK), jnp.int32),
            pltpu.VMEM((2, CHUNK), jnp.int32),
            pltpu.VMEM((2, CHUNK, D), jnp.float32),
            pltpu.VMEM((E_TAIL,), jnp.int32),
            pltpu.VMEM((E_TAIL,), jnp.int32),
            pltpu.VMEM((E_TAIL, D), jnp.float32),
            pltpu.SemaphoreType.DMA((2,)),
            pltpu.SemaphoreType.DMA((2,)),
        ],
    )
    def k(y_hbm, src_hbm, dst_hbm, out_hbm, acc_sh, sidx_v, didx_v,
          rows_v, tsidx_v, tdidx_v, trows_v, isem, gsem):
        cid = lax.axis_index("c")
        sid = lax.axis_index("s")

        def fill_zeros(i, carry):
            for j in range(D // 16):
                rows_v[0, i, pl.ds(j * 16, 16)] = jnp.zeros((16,), jnp.float32)
            return carry

        lax.fori_loop(0, CHUNK, fill_zeros, 0)
        row0 = pl.multiple_of(sid * ROWS_PER_TILE, 8)
        for r in range(4):
            pltpu.sync_copy(rows_v.at[0],
                            acc_sh.at[pl.ds(row0 + r * CHUNK, CHUNK)])
        pltpu.sync_copy(rows_v.at[0, pl.ds(0, ROWS_PER_TILE - 4 * CHUNK)],
                        acc_sh.at[pl.ds(row0 + 4 * CHUNK,
                                        ROWS_PER_TILE - 4 * CHUNK)])

        @pl.when(sid == NS - 1)
        def _():
            pltpu.sync_copy(rows_v.at[0, pl.ds(0, ROWS_TAIL)],
                            acc_sh.at[pl.ds(NS * ROWS_PER_TILE, ROWS_TAIL)])

        plsc.subcore_barrier()

        base0 = (cid * NS + sid) * DEG_EDGES_PER_WORKER

        def idx_issue(i, b):
            base = base0 + i * CHUNK
            pltpu.async_copy(src_hbm.at[pl.ds(base, CHUNK)], sidx_v.at[b],
                             isem.at[b])
            pltpu.async_copy(dst_hbm.at[pl.ds(base, CHUNK)], didx_v.at[b],
                             isem.at[b])

        def idx_wait(i, b):
            base = base0 + i * CHUNK
            pltpu.make_async_copy(src_hbm.at[pl.ds(base, CHUNK)], sidx_v.at[b],
                                  isem.at[b]).wait()
            pltpu.make_async_copy(dst_hbm.at[pl.ds(base, CHUNK)], didx_v.at[b],
                                  isem.at[b]).wait()

        def gather_issue(b):
            pltpu.async_copy(y_hbm.at[sidx_v.at[b]], rows_v.at[b], gsem.at[b])

        def gather_wait(b):
            pltpu.make_async_copy(y_hbm.at[sidx_v.at[b]], rows_v.at[b],
                                  gsem.at[b]).wait()

        def scatter(b):
            pltpu.sync_copy(rows_v.at[b], acc_sh.at[didx_v.at[b]], add=True)

        tbase = base0 + E_NFULL * CHUNK
        tsrc_slice = src_hbm.at[pl.ds(tbase, E_TAIL)]
        tdst_slice = dst_hbm.at[pl.ds(tbase, E_TAIL)]

        idx_issue(0, 0)
        idx_issue(1, 1)
        idx_wait(0, 0)
        gather_issue(0)

        def body(t, carry):
            i0 = 2 * t
            idx_wait(i0 + 1, 1)
            gather_issue(1)
            gather_wait(0)
            idx_issue(i0 + 2, 0)
            scatter(0)
            idx_wait(i0 + 2, 0)
            gather_issue(0)
            gather_wait(1)
            idx_issue(i0 + 3, 1)
            scatter(1)
            return carry

        lax.fori_loop(0, E_NFULL // 2 - 1, body, 0)
        idx_wait(E_NFULL - 1, 1)
        gather_issue(1)
        gather_wait(0)
        pltpu.async_copy(tsrc_slice, tsidx_v, isem.at[0])
        pltpu.async_copy(tdst_slice, tdidx_v, isem.at[0])
        scatter(0)
        pltpu.make_async_copy(tsrc_slice, tsidx_v, isem.at[0]).wait()
        pltpu.make_async_copy(tdst_slice, tdidx_v, isem.at[0]).wait()
        pltpu.async_copy(y_hbm.at[tsidx_v], trows_v, gsem.at[0])
        gather_wait(1)
        scatter(1)
        pltpu.make_async_copy(y_hbm.at[tsidx_v], trows_v, gsem.at[0]).wait()
        pltpu.sync_copy(trows_v, acc_sh.at[tdidx_v], add=True)
        plsc.subcore_barrier()

        obase = pl.multiple_of(cid * N + row0, 8)
        for r in range(2):
            pltpu.async_copy(acc_sh.at[pl.ds(row0 + r * CHUNK, CHUNK)],
                             rows_v.at[r], gsem.at[r])
        for r in range(4):
            b = r % 2
            pltpu.make_async_copy(acc_sh.at[pl.ds(row0 + r * CHUNK, CHUNK)],
                                  rows_v.at[b], gsem.at[b]).wait()
            pltpu.sync_copy(rows_v.at[b],
                            out_hbm.at[pl.ds(obase + r * CHUNK, CHUNK)])
            if r < 2:
                pltpu.async_copy(
                    acc_sh.at[pl.ds(row0 + (r + 2) * CHUNK, CHUNK)],
                    rows_v.at[b], gsem.at[b])
        rem = ROWS_PER_TILE - 4 * CHUNK
        pltpu.sync_copy(acc_sh.at[pl.ds(row0 + 4 * CHUNK, rem)],
                        rows_v.at[0, pl.ds(0, rem)])
        pltpu.sync_copy(rows_v.at[0, pl.ds(0, rem)],
                        out_hbm.at[pl.ds(obase + 4 * CHUNK, rem)])

        @pl.when(sid == NS - 1)
        def _():
            tb = pl.multiple_of(NS * ROWS_PER_TILE, 8)
            pltpu.sync_copy(acc_sh.at[pl.ds(tb, ROWS_TAIL)],
                            rows_v.at[1, pl.ds(0, ROWS_TAIL)])
            pltpu.sync_copy(rows_v.at[1, pl.ds(0, ROWS_TAIL)],
                            out_hbm.at[pl.ds(pl.multiple_of(cid * N + tb, 8),
                                             ROWS_TAIL)])

    return k(y, src, dst)


_BN = 1000


def _prescale_body(x_ref, w_ref, d0_ref, d1_ref, y_ref):
    deg = d0_ref[...] + d1_ref[...] + 1.0
    dis = lax.rsqrt(deg)
    xw = jnp.dot(x_ref[...], w_ref[...], preferred_element_type=jnp.float32)
    y_ref[...] = xw * dis


def _prescale(x, w, deg0, deg1):
    return pl.pallas_call(
        _prescale_body,
        grid=(N // _BN,),
        in_specs=[
            pl.BlockSpec((_BN, D), lambda i: (i, 0)),
            pl.BlockSpec((D, D), lambda i: (0, 0)),
            pl.BlockSpec((_BN, 1), lambda i: (i, 0)),
            pl.BlockSpec((_BN, 1), lambda i: (i, 0)),
        ],
        out_specs=pl.BlockSpec((_BN, D), lambda i: (i, 0)),
        out_shape=jax.ShapeDtypeStruct((N, D), jnp.float32),
    )(x, w, deg0, deg1)


def _finish_body(a0_ref, a1_ref, y_ref, d0_ref, d1_ref, bc_ref, wl_ref, bl_ref,
                 h_ref, o_ref):
    deg = d0_ref[...] + d1_ref[...] + 1.0
    dis = lax.rsqrt(deg)
    s = (a0_ref[...] + a1_ref[...] + y_ref[...]) * dis + bc_ref[...]
    h = jnp.maximum(s, 0.0)
    h_ref[...] = h
    z = jnp.dot(h, wl_ref[...], preferred_element_type=jnp.float32) + bl_ref[...]
    o_ref[...] = jax.nn.sigmoid(z)


def _finish(acc2, y, deg0, deg1, bc, wl, bl):
    nb = N // _BN
    return pl.pallas_call(
        _finish_body,
        grid=(N // _BN,),
        in_specs=[
            pl.BlockSpec((_BN, D), lambda i: (i, 0)),
            pl.BlockSpec((_BN, D), lambda i: (nb + i, 0)),
            pl.BlockSpec((_BN, D), lambda i: (i, 0)),
            pl.BlockSpec((_BN, 1), lambda i: (i, 0)),
            pl.BlockSpec((_BN, 1), lambda i: (i, 0)),
            pl.BlockSpec((1, D), lambda i: (0, 0)),
            pl.BlockSpec((D, 1), lambda i: (0, 0)),
            pl.BlockSpec((1, 1), lambda i: (0, 0)),
        ],
        out_specs=[
            pl.BlockSpec((_BN, D), lambda i: (i, 0)),
            pl.BlockSpec((_BN, 1), lambda i: (i, 0)),
        ],
        out_shape=[
            jax.ShapeDtypeStruct((N, D), jnp.float32),
            jax.ShapeDtypeStruct((N, 1), jnp.float32),
        ],
    )(acc2, acc2, y, deg0, deg1, bc, wl, bl)


def kernel(x, edge_index, W_conv, b_conv, W_lin, b_lin):
    src = edge_index[0].astype(jnp.int32)
    dst = edge_index[1].astype(jnp.int32)

    degw = _deg_kernel(dst)
    deg0 = degw[:N].reshape(N, 1)
    deg1 = degw[N:].reshape(N, 1)

    y = _prescale(x, W_conv, deg0, deg1)
    acc2 = _edge_kernel(y, src, dst)

    h, out = _finish(acc2, y, deg0, deg1,
                     b_conv.reshape(1, D), W_lin, b_lin.reshape(1, 1))
    return (out, h)


# speedup vs baseline: 41.0106x; 1.0325x over previous
import functools

import jax
import jax.numpy as jnp
from jax import lax
from jax.experimental import pallas as pl
from jax.experimental.pallas import tpu as pltpu
from jax.experimental.pallas import tpu_sc as plsc

N = 10000
E = 320000
D = 128

NC = 2
NS = 16
ROWS_PER_TILE = 624
ROWS_TAIL = N - NS * ROWS_PER_TILE
CHUNK = 128

DEG_EDGES_PER_WORKER = E // (NC * NS)
DEG_NFULL = DEG_EDGES_PER_WORKER // CHUNK
DEG_TAIL = DEG_EDGES_PER_WORKER - DEG_NFULL * CHUNK


def _deg_kernel(dst):
    @functools.partial(
        pl.kernel,
        out_type=jax.ShapeDtypeStruct((NC * N,), jnp.float32),
        mesh=plsc.VectorSubcoreMesh(core_axis_name="c", subcore_axis_name="s"),
        scratch_types=[
            pltpu.VMEM_SHARED((N,), jnp.float32),
            pltpu.VMEM((ROWS_PER_TILE,), jnp.float32),
            pltpu.VMEM((3, CHUNK), jnp.int32),
            pltpu.VMEM((CHUNK,), jnp.float32),
            pltpu.VMEM((DEG_TAIL,), jnp.int32),
            pltpu.SemaphoreType.DMA((3,)),
        ],
    )
    def k(dst_hbm, out_hbm, acc_sh, zeros_v, idx_v, ones_v, tidx_v, isem):
        cid = lax.axis_index("c")
        sid = lax.axis_index("s")

        def fill_zeros(i, carry):
            zeros_v[pl.ds(i * 16, 16)] = jnp.zeros((16,), jnp.float32)
            return carry

        lax.fori_loop(0, ROWS_PER_TILE // 16, fill_zeros, 0)

        def fill_ones(i, carry):
            ones_v[pl.ds(i * 16, 16)] = jnp.ones((16,), jnp.float32)
            return carry

        lax.fori_loop(0, CHUNK // 16, fill_ones, 0)

        row0 = pl.multiple_of(sid * ROWS_PER_TILE, 8)
        pltpu.sync_copy(zeros_v, acc_sh.at[pl.ds(row0, ROWS_PER_TILE)])

        @pl.when(sid == NS - 1)
        def _():
            pltpu.sync_copy(zeros_v.at[pl.ds(0, ROWS_TAIL)],
                            acc_sh.at[pl.ds(NS * ROWS_PER_TILE, ROWS_TAIL)])

        plsc.subcore_barrier()

        base0 = (cid * NS + sid) * DEG_EDGES_PER_WORKER

        def issue(i, b):
            pltpu.async_copy(dst_hbm.at[pl.ds(base0 + i * CHUNK, CHUNK)],
                             idx_v.at[b], isem.at[b])

        def wait(i, b):
            pltpu.make_async_copy(dst_hbm.at[pl.ds(base0 + i * CHUNK, CHUNK)],
                                  idx_v.at[b], isem.at[b]).wait()

        def scatter(b):
            pltpu.sync_copy(ones_v, acc_sh.at[idx_v.at[b]], add=True)

        tslice = dst_hbm.at[pl.ds(base0 + DEG_NFULL * CHUNK, DEG_TAIL)]

        issue(0, 0)
        issue(1, 1)
        issue(2, 2)

        def body(t, carry):
            i = 3 * t
            for b in range(3):
                wait(i + b, b)
                scatter(b)
                issue(i + b + 3, b)
            return carry

        lax.fori_loop(0, DEG_NFULL // 3 - 1, body, 0)
        wait(DEG_NFULL - 3, 0)
        scatter(0)
        pltpu.async_copy(tslice, tidx_v, isem.at[0])
        wait(DEG_NFULL - 2, 1)
        scatter(1)
        wait(DEG_NFULL - 1, 2)
        scatter(2)
        pltpu.make_async_copy(tslice, tidx_v, isem.at[0]).wait()
        pltpu.sync_copy(ones_v.at[pl.ds(0, DEG_TAIL)], acc_sh.at[tidx_v],
                        add=True)
        plsc.subcore_barrier()
        obase = pl.multiple_of(cid * N + row0, 8)
        pltpu.sync_copy(acc_sh.at[pl.ds(row0, ROWS_PER_TILE)], zeros_v)
        pltpu.sync_copy(zeros_v, out_hbm.at[pl.ds(obase, ROWS_PER_TILE)])

        @pl.when(sid == NS - 1)
        def _():
            pltpu.sync_copy(acc_sh.at[pl.ds(NS * ROWS_PER_TILE, ROWS_TAIL)],
                            zeros_v.at[pl.ds(0, ROWS_TAIL)])
            pltpu.sync_copy(
                zeros_v.at[pl.ds(0, ROWS_TAIL)],
                out_hbm.at[pl.ds(pl.multiple_of(cid * N + NS * ROWS_PER_TILE, 8),
                                 ROWS_TAIL)])

    return k(dst)


ECHUNK = 80
E_NCHUNK = DEG_EDGES_PER_WORKER // ECHUNK
NBUF = 4


def _edge_kernel(y, src, dst):
    @functools.partial(
        pl.kernel,
        out_type=jax.ShapeDtypeStruct((NC * N, D), jnp.float32),
        mesh=plsc.VectorSubcoreMesh(core_axis_name="c", subcore_axis_name="s"),
        scratch_types=[
            pltpu.VMEM_SHARED((N, D), jnp.float32),
            pltpu.VMEM((NBUF, ECHUNK), jnp.int32),
            pltpu.VMEM((NBUF, ECHUNK), jnp.int32),
            pltpu.VMEM((NBUF, ECHUNK), jnp.int32),
            pltpu.VMEM((NBUF, ECHUNK, D), jnp.float32),
            pltpu.SemaphoreType.DMA((NBUF,)),
            pltpu.SemaphoreType.DMA((NBUF,)),
            pltpu.SemaphoreType.DMA((NBUF,)),
        ],
    )
    def k(y_hbm, src_hbm, dst_hbm, out_hbm, acc_sh, sidx_v, didx_v, sdidx_v,
          rows_v, isem, gsem, ssem):
        cid = lax.axis_index("c")
        sid = lax.axis_index("s")

        def fill_zeros(i, carry):
            for j in range(D // 16):
                rows_v[0, i, pl.ds(j * 16, 16)] = jnp.zeros((16,), jnp.float32)
            return carry

        lax.fori_loop(0, ECHUNK, fill_zeros, 0)
        row0 = pl.multiple_of(sid * ROWS_PER_TILE, 8)
        for r in range(7):
            pltpu.sync_copy(rows_v.at[0],
                            acc_sh.at[pl.ds(row0 + r * ECHUNK, ECHUNK)])
        pltpu.sync_copy(rows_v.at[0, pl.ds(0, ROWS_PER_TILE - 7 * ECHUNK)],
                        acc_sh.at[pl.ds(row0 + 7 * ECHUNK,
                                        ROWS_PER_TILE - 7 * ECHUNK)])

        @pl.when(sid == NS - 1)
        def _():
            pltpu.sync_copy(rows_v.at[0, pl.ds(0, ROWS_TAIL)],
                            acc_sh.at[pl.ds(NS * ROWS_PER_TILE, ROWS_TAIL)])

        plsc.subcore_barrier()

        base0 = (cid * NS + sid) * DEG_EDGES_PER_WORKER
        LAST = E_NCHUNK - 1

        def idx_issue(i, b):
            base = base0 + i * ECHUNK
            pltpu.async_copy(src_hbm.at[pl.ds(base, ECHUNK)], sidx_v.at[b],
                             isem.at[b])
            pltpu.async_copy(dst_hbm.at[pl.ds(base, ECHUNK)], didx_v.at[b],
                             isem.at[b])

        def idx_wait(i, b):
            base = base0 + i * ECHUNK
            pltpu.make_async_copy(src_hbm.at[pl.ds(base, ECHUNK)], sidx_v.at[b],
                                  isem.at[b]).wait()
            pltpu.make_async_copy(dst_hbm.at[pl.ds(base, ECHUNK)], didx_v.at[b],
                                  isem.at[b]).wait()

        def dcopy(b):
            for j in range(ECHUNK // 16):
                sdidx_v[b, pl.ds(j * 16, 16)] = didx_v[b, pl.ds(j * 16, 16)]

        def gather_issue(b):
            pltpu.async_copy(y_hbm.at[sidx_v.at[b]], rows_v.at[b], gsem.at[b])

        def gather_wait(b):
            pltpu.make_async_copy(y_hbm.at[sidx_v.at[b]], rows_v.at[b],
                                  gsem.at[b]).wait()

        def scatter_issue(b):
            pltpu.async_copy(rows_v.at[b], acc_sh.at[sdidx_v.at[b]], ssem.at[b],
                             add=True)

        def scatter_wait(b):
            pltpu.make_async_copy(rows_v.at[b], acc_sh.at[sdidx_v.at[b]],
                                  ssem.at[b]).wait()

        def prep(i, b):
            idx_wait(i, b)
            dcopy(b)
            gather_issue(b)

        for i in range(NBUF):
            idx_issue(i, i)
        prep(0, 0)
        prep(1, 1)
        gather_wait(0)
        idx_issue(NBUF, 0)
        scatter_issue(0)
        prep(2, 2)
        gather_wait(1)
        idx_issue(NBUF + 1, 1)
        scatter_issue(1)
        prep(3, 3)

        def body(t, carry):
            j0 = 4 * t + 2
            for u in range(4):
                j = j0 + u
                b = (2 + u) % NBUF
                scatter_wait((2 + u + 2) % NBUF)
                prep(j + 2, (2 + u + 2) % NBUF)
                gather_wait(b)
                idx_issue(j + 4, b)
                scatter_issue(b)
            return carry

        lax.fori_loop(0, 29, body, 0)

        for j in range(118, 125):
            scatter_wait((j - 2) % NBUF)
            if j + 2 <= LAST:
                prep(j + 2, (j + 2) % NBUF)
            gather_wait(j % NBUF)
            if j + 4 <= LAST:
                idx_issue(j + 4, j % NBUF)
            scatter_issue(j % NBUF)
        scatter_wait(123 % NBUF)
        scatter_wait(124 % NBUF)
        plsc.subcore_barrier()

        obase = pl.multiple_of(cid * N + row0, 8)

        def cp_size(r):
            return ECHUNK if r < 7 else ROWS_PER_TILE - 7 * ECHUNK

        def cp_in(r, b):
            pltpu.async_copy(acc_sh.at[pl.ds(row0 + r * ECHUNK, cp_size(r))],
                             rows_v.at[b, pl.ds(0, cp_size(r))], gsem.at[b])

        def cp_in_wait(r, b):
            pltpu.make_async_copy(
                acc_sh.at[pl.ds(row0 + r * ECHUNK, cp_size(r))],
                rows_v.at[b, pl.ds(0, cp_size(r))], gsem.at[b]).wait()

        for r in range(NBUF):
            cp_in(r, r)
        for r in range(8):
            b = r % NBUF
            cp_in_wait(r, b)
            pltpu.sync_copy(rows_v.at[b, pl.ds(0, cp_size(r))],
                            out_hbm.at[pl.ds(obase + r * ECHUNK, cp_size(r))])
            if r + NBUF < 8:
                cp_in(r + NBUF, b)

        @pl.when(sid == NS - 1)
        def _():
            tb = pl.multiple_of(NS * ROWS_PER_TILE, 8)
            pltpu.sync_copy(acc_sh.at[pl.ds(tb, ROWS_TAIL)],
                            rows_v.at[0, pl.ds(0, ROWS_TAIL)])
            pltpu.sync_copy(rows_v.at[0, pl.ds(0, ROWS_TAIL)],
                            out_hbm.at[pl.ds(pl.multiple_of(cid * N + tb, 8),
                                             ROWS_TAIL)])

    return k(y, src, dst)


_BN = 1000


def _prescale_body(x_ref, w_ref, d0_ref, d1_ref, y_ref):
    deg = d0_ref[...] + d1_ref[...] + 1.0
    dis = lax.rsqrt(deg)
    xw = jnp.dot(x_ref[...], w_ref[...], preferred_element_type=jnp.float32)
    y_ref[...] = xw * dis


def _prescale(x, w, deg0, deg1):
    return pl.pallas_call(
        _prescale_body,
        grid=(N // _BN,),
        in_specs=[
            pl.BlockSpec((_BN, D), lambda i: (i, 0)),
            pl.BlockSpec((D, D), lambda i: (0, 0)),
            pl.BlockSpec((_BN, 1), lambda i: (i, 0)),
            pl.BlockSpec((_BN, 1), lambda i: (i, 0)),
        ],
        out_specs=pl.BlockSpec((_BN, D), lambda i: (i, 0)),
        out_shape=jax.ShapeDtypeStruct((N, D), jnp.float32),
    )(x, w, deg0, deg1)


def _finish_body(a0_ref, a1_ref, y_ref, d0_ref, d1_ref, bc_ref, wl_ref, bl_ref,
                 h_ref, o_ref):
    deg = d0_ref[...] + d1_ref[...] + 1.0
    dis = lax.rsqrt(deg)
    s = (a0_ref[...] + a1_ref[...] + y_ref[...]) * dis + bc_ref[...]
    h = jnp.maximum(s, 0.0)
    h_ref[...] = h
    z = jnp.dot(h, wl_ref[...], preferred_element_type=jnp.float32) + bl_ref[...]
    o_ref[...] = jax.nn.sigmoid(z)


def _finish(acc2, y, deg0, deg1, bc, wl, bl):
    nb = N // _BN
    return pl.pallas_call(
        _finish_body,
        grid=(N // _BN,),
        in_specs=[
            pl.BlockSpec((_BN, D), lambda i: (i, 0)),
            pl.BlockSpec((_BN, D), lambda i: (nb + i, 0)),
            pl.BlockSpec((_BN, D), lambda i: (i, 0)),
            pl.BlockSpec((_BN, 1), lambda i: (i, 0)),
            pl.BlockSpec((_BN, 1), lambda i: (i, 0)),
            pl.BlockSpec((1, D), lambda i: (0, 0)),
            pl.BlockSpec((D, 1), lambda i: (0, 0)),
            pl.BlockSpec((1, 1), lambda i: (0, 0)),
        ],
        out_specs=[
            pl.BlockSpec((_BN, D), lambda i: (i, 0)),
            pl.BlockSpec((_BN, 1), lambda i: (i, 0)),
        ],
        out_shape=[
            jax.ShapeDtypeStruct((N, D), jnp.float32),
            jax.ShapeDtypeStruct((N, 1), jnp.float32),
        ],
    )(acc2, acc2, y, deg0, deg1, bc, wl, bl)


def kernel(x, edge_index, W_conv, b_conv, W_lin, b_lin):
    src = edge_index[0].astype(jnp.int32)
    dst = edge_index[1].astype(jnp.int32)

    degw = _deg_kernel(dst)
    deg0 = degw[:N].reshape(N, 1)
    deg1 = degw[N:].reshape(N, 1)

    y = _prescale(x, W_conv, deg0, deg1)
    acc2 = _edge_kernel(y, src, dst)

    h, out = _finish(acc2, y, deg0, deg1,
                     b_conv.reshape(1, D), W_lin, b_lin.reshape(1, 1))
    return (out, h)


# speedup vs baseline: 45.4186x; 1.1075x over previous
import functools

import jax
import jax.numpy as jnp
from jax import lax
from jax.experimental import pallas as pl
from jax.experimental.pallas import tpu as pltpu
from jax.experimental.pallas import tpu_sc as plsc

N = 10000
E = 320000
D = 128

NC = 2
NS = 16
ROWS_PER_TILE = 624
ROWS_TAIL = N - NS * ROWS_PER_TILE
CHUNK = 128

DEG_EDGES_PER_WORKER = E // (NC * NS)
DEG_NFULL = DEG_EDGES_PER_WORKER // CHUNK
DEG_TAIL = DEG_EDGES_PER_WORKER - DEG_NFULL * CHUNK


def _deg_kernel(ei_flat):
    @functools.partial(
        pl.kernel,
        out_type=jax.ShapeDtypeStruct((8 * N,), jnp.float32),
        mesh=plsc.VectorSubcoreMesh(core_axis_name="c", subcore_axis_name="s"),
        scratch_types=[
            pltpu.VMEM_SHARED((N,), jnp.float32),
            pltpu.VMEM((ROWS_PER_TILE,), jnp.float32),
            pltpu.VMEM((3, CHUNK), jnp.int32),
            pltpu.VMEM((CHUNK,), jnp.float32),
            pltpu.VMEM((DEG_TAIL,), jnp.int32),
            pltpu.SemaphoreType.DMA((3,)),
        ],
    )
    def k(dst_hbm, out_hbm, acc_sh, zeros_v, idx_v, ones_v, tidx_v, isem):
        cid = lax.axis_index("c")
        sid = lax.axis_index("s")

        def fill_zeros(i, carry):
            zeros_v[pl.ds(i * 16, 16)] = jnp.zeros((16,), jnp.float32)
            return carry

        lax.fori_loop(0, ROWS_PER_TILE // 16, fill_zeros, 0)

        def fill_ones(i, carry):
            ones_v[pl.ds(i * 16, 16)] = jnp.ones((16,), jnp.float32)
            return carry

        lax.fori_loop(0, CHUNK // 16, fill_ones, 0)

        row0 = pl.multiple_of(sid * ROWS_PER_TILE, 8)
        pltpu.sync_copy(zeros_v, acc_sh.at[pl.ds(row0, ROWS_PER_TILE)])

        @pl.when(sid == NS - 1)
        def _():
            pltpu.sync_copy(zeros_v.at[pl.ds(0, ROWS_TAIL)],
                            acc_sh.at[pl.ds(NS * ROWS_PER_TILE, ROWS_TAIL)])

        plsc.subcore_barrier()

        base0 = E + (cid * NS + sid) * DEG_EDGES_PER_WORKER

        def issue(i, b):
            pltpu.async_copy(dst_hbm.at[pl.ds(base0 + i * CHUNK, CHUNK)],
                             idx_v.at[b], isem.at[b])

        def wait(i, b):
            pltpu.make_async_copy(dst_hbm.at[pl.ds(base0 + i * CHUNK, CHUNK)],
                                  idx_v.at[b], isem.at[b]).wait()

        def scatter(b):
            pltpu.sync_copy(ones_v, acc_sh.at[idx_v.at[b]], add=True)

        tslice = dst_hbm.at[pl.ds(base0 + DEG_NFULL * CHUNK, DEG_TAIL)]

        issue(0, 0)
        issue(1, 1)
        issue(2, 2)

        def body(t, carry):
            i = 3 * t
            for b in range(3):
                wait(i + b, b)
                scatter(b)
                issue(i + b + 3, b)
            return carry

        lax.fori_loop(0, DEG_NFULL // 3 - 1, body, 0)
        wait(DEG_NFULL - 3, 0)
        scatter(0)
        pltpu.async_copy(tslice, tidx_v, isem.at[0])
        wait(DEG_NFULL - 2, 1)
        scatter(1)
        wait(DEG_NFULL - 1, 2)
        scatter(2)
        pltpu.make_async_copy(tslice, tidx_v, isem.at[0]).wait()
        pltpu.sync_copy(ones_v.at[pl.ds(0, DEG_TAIL)], acc_sh.at[tidx_v],
                        add=True)
        plsc.subcore_barrier()
        obase = pl.multiple_of(cid * N + row0, 8)
        pltpu.sync_copy(acc_sh.at[pl.ds(row0, ROWS_PER_TILE)], zeros_v)
        pltpu.sync_copy(zeros_v, out_hbm.at[pl.ds(obase, ROWS_PER_TILE)])

        @pl.when(sid == NS - 1)
        def _():
            pltpu.sync_copy(acc_sh.at[pl.ds(NS * ROWS_PER_TILE, ROWS_TAIL)],
                            zeros_v.at[pl.ds(0, ROWS_TAIL)])
            pltpu.sync_copy(
                zeros_v.at[pl.ds(0, ROWS_TAIL)],
                out_hbm.at[pl.ds(pl.multiple_of(cid * N + NS * ROWS_PER_TILE, 8),
                                 ROWS_TAIL)])

    return k(ei_flat)


ECHUNK = 80
E_NCHUNK = DEG_EDGES_PER_WORKER // ECHUNK
NBUF = 4


def _edge_kernel(y, ei_flat):
    @functools.partial(
        pl.kernel,
        out_type=jax.ShapeDtypeStruct((NC * N, D), jnp.float32),
        mesh=plsc.VectorSubcoreMesh(core_axis_name="c", subcore_axis_name="s"),
        scratch_types=[
            pltpu.VMEM_SHARED((N, D), jnp.float32),
            pltpu.VMEM((NBUF, ECHUNK), jnp.int32),
            pltpu.VMEM((NBUF, ECHUNK), jnp.int32),
            pltpu.VMEM((NBUF, ECHUNK), jnp.int32),
            pltpu.VMEM((NBUF, ECHUNK, D), jnp.float32),
            pltpu.SemaphoreType.DMA((NBUF,)),
            pltpu.SemaphoreType.DMA((NBUF,)),
            pltpu.SemaphoreType.DMA((NBUF,)),
        ],
    )
    def k(y_hbm, ei_hbm, out_hbm, acc_sh, sidx_v, didx_v, sdidx_v,
          rows_v, isem, gsem, ssem):
        cid = lax.axis_index("c")
        sid = lax.axis_index("s")

        def fill_zeros(i, carry):
            for j in range(D // 16):
                rows_v[0, i, pl.ds(j * 16, 16)] = jnp.zeros((16,), jnp.float32)
            return carry

        lax.fori_loop(0, ECHUNK, fill_zeros, 0)
        row0 = pl.multiple_of(sid * ROWS_PER_TILE, 8)
        for r in range(7):
            pltpu.sync_copy(rows_v.at[0],
                            acc_sh.at[pl.ds(row0 + r * ECHUNK, ECHUNK)])
        pltpu.sync_copy(rows_v.at[0, pl.ds(0, ROWS_PER_TILE - 7 * ECHUNK)],
                        acc_sh.at[pl.ds(row0 + 7 * ECHUNK,
                                        ROWS_PER_TILE - 7 * ECHUNK)])

        @pl.when(sid == NS - 1)
        def _():
            pltpu.sync_copy(rows_v.at[0, pl.ds(0, ROWS_TAIL)],
                            acc_sh.at[pl.ds(NS * ROWS_PER_TILE, ROWS_TAIL)])

        plsc.subcore_barrier()

        base0 = (cid * NS + sid) * DEG_EDGES_PER_WORKER
        LAST = E_NCHUNK - 1

        def idx_issue(i, b):
            base = base0 + i * ECHUNK
            pltpu.async_copy(ei_hbm.at[pl.ds(base, ECHUNK)], sidx_v.at[b],
                             isem.at[b])
            pltpu.async_copy(ei_hbm.at[pl.ds(E + base, ECHUNK)], didx_v.at[b],
                             isem.at[b])

        def idx_wait(i, b):
            base = base0 + i * ECHUNK
            pltpu.make_async_copy(ei_hbm.at[pl.ds(base, ECHUNK)], sidx_v.at[b],
                                  isem.at[b]).wait()
            pltpu.make_async_copy(ei_hbm.at[pl.ds(E + base, ECHUNK)],
                                  didx_v.at[b], isem.at[b]).wait()

        def dcopy(b):
            for j in range(ECHUNK // 16):
                sdidx_v[b, pl.ds(j * 16, 16)] = didx_v[b, pl.ds(j * 16, 16)]

        def gather_issue(b):
            pltpu.async_copy(y_hbm.at[sidx_v.at[b]], rows_v.at[b], gsem.at[b])

        def gather_wait(b):
            pltpu.make_async_copy(y_hbm.at[sidx_v.at[b]], rows_v.at[b],
                                  gsem.at[b]).wait()

        def scatter_issue(b):
            pltpu.async_copy(rows_v.at[b], acc_sh.at[sdidx_v.at[b]], ssem.at[b],
                             add=True)

        def scatter_wait(b):
            pltpu.make_async_copy(rows_v.at[b], acc_sh.at[sdidx_v.at[b]],
                                  ssem.at[b]).wait()

        def prep(i, b):
            idx_wait(i, b)
            dcopy(b)
            gather_issue(b)

        for i in range(NBUF):
            idx_issue(i, i)
        prep(0, 0)
        prep(1, 1)
        gather_wait(0)
        idx_issue(NBUF, 0)
        scatter_issue(0)
        prep(2, 2)
        gather_wait(1)
        idx_issue(NBUF + 1, 1)
        scatter_issue(1)
        prep(3, 3)

        def body(t, carry):
            j0 = 4 * t + 2
            for u in range(4):
                j = j0 + u
                b = (2 + u) % NBUF
                scatter_wait((2 + u + 2) % NBUF)
                prep(j + 2, (2 + u + 2) % NBUF)
                gather_wait(b)
                idx_issue(j + 4, b)
                scatter_issue(b)
            return carry

        lax.fori_loop(0, 29, body, 0)

        for j in range(118, 125):
            scatter_wait((j - 2) % NBUF)
            if j + 2 <= LAST:
                prep(j + 2, (j + 2) % NBUF)
            gather_wait(j % NBUF)
            if j + 4 <= LAST:
                idx_issue(j + 4, j % NBUF)
            scatter_issue(j % NBUF)
        scatter_wait(123 % NBUF)
        scatter_wait(124 % NBUF)
        plsc.subcore_barrier()

        obase = pl.multiple_of(cid * N + row0, 8)

        def cp_size(r):
            return ECHUNK if r < 7 else ROWS_PER_TILE - 7 * ECHUNK

        def cp_in(r, b):
            pltpu.async_copy(acc_sh.at[pl.ds(row0 + r * ECHUNK, cp_size(r))],
                             rows_v.at[b, pl.ds(0, cp_size(r))], gsem.at[b])

        def cp_in_wait(r, b):
            pltpu.make_async_copy(
                acc_sh.at[pl.ds(row0 + r * ECHUNK, cp_size(r))],
                rows_v.at[b, pl.ds(0, cp_size(r))], gsem.at[b]).wait()

        for r in range(NBUF):
            cp_in(r, r)
        for r in range(8):
            b = r % NBUF
            cp_in_wait(r, b)
            pltpu.sync_copy(rows_v.at[b, pl.ds(0, cp_size(r))],
                            out_hbm.at[pl.ds(obase + r * ECHUNK, cp_size(r))])
            if r + NBUF < 8:
                cp_in(r + NBUF, b)

        @pl.when(sid == NS - 1)
        def _():
            tb = pl.multiple_of(NS * ROWS_PER_TILE, 8)
            pltpu.sync_copy(acc_sh.at[pl.ds(tb, ROWS_TAIL)],
                            rows_v.at[0, pl.ds(0, ROWS_TAIL)])
            pltpu.sync_copy(rows_v.at[0, pl.ds(0, ROWS_TAIL)],
                            out_hbm.at[pl.ds(pl.multiple_of(cid * N + tb, 8),
                                             ROWS_TAIL)])

    return k(y, ei_flat)


_BN = 1000


def _dis_body(d_ref, o_ref):
    dsum = d_ref[0:1, :] + d_ref[1:2, :] + 1.0
    o_ref[...] = lax.transpose(lax.rsqrt(dsum), (1, 0))


def _dis_kernel(deg8):
    return pl.pallas_call(
        _dis_body,
        grid=(1,),
        in_specs=[pl.BlockSpec((8, N), lambda i: (0, 0))],
        out_specs=pl.BlockSpec((N, 1), lambda i: (0, 0)),
        out_shape=jax.ShapeDtypeStruct((N, 1), jnp.float32),
    )(deg8)


def _prescale_body(x_ref, w_ref, d_ref, y_ref):
    xw = jnp.dot(x_ref[...], w_ref[...], preferred_element_type=jnp.float32)
    y_ref[...] = xw * d_ref[...]


def _prescale(x, w, dis):
    return pl.pallas_call(
        _prescale_body,
        grid=(N // _BN,),
        in_specs=[
            pl.BlockSpec((_BN, D), lambda i: (i, 0)),
            pl.BlockSpec((D, D), lambda i: (0, 0)),
            pl.BlockSpec((_BN, 1), lambda i: (i, 0)),
        ],
        out_specs=pl.BlockSpec((_BN, D), lambda i: (i, 0)),
        out_shape=jax.ShapeDtypeStruct((N, D), jnp.float32),
    )(x, w, dis)


def _finish_body(a0_ref, a1_ref, y_ref, d_ref, bc_ref, wl_ref, bl_ref,
                 h_ref, o_ref):
    dis = d_ref[...]
    s = (a0_ref[...] + a1_ref[...] + y_ref[...]) * dis + bc_ref[...]
    h = jnp.maximum(s, 0.0)
    h_ref[...] = h
    z = jnp.dot(h, wl_ref[...], preferred_element_type=jnp.float32) + bl_ref[...]
    o_ref[...] = jax.nn.sigmoid(z)


def _finish(acc2, y, dis, bc, wl, bl):
    nb = N // _BN
    return pl.pallas_call(
        _finish_body,
        grid=(N // _BN,),
        in_specs=[
            pl.BlockSpec((_BN, D), lambda i: (i, 0)),
            pl.BlockSpec((_BN, D), lambda i: (nb + i, 0)),
            pl.BlockSpec((_BN, D), lambda i: (i, 0)),
            pl.BlockSpec((_BN, 1), lambda i: (i, 0)),
            pl.BlockSpec((1, D), lambda i: (0, 0)),
            pl.BlockSpec((D, 1), lambda i: (0, 0)),
            pl.BlockSpec((1, 1), lambda i: (0, 0)),
        ],
        out_specs=[
            pl.BlockSpec((_BN, D), lambda i: (i, 0)),
            pl.BlockSpec((_BN, 1), lambda i: (i, 0)),
        ],
        out_shape=[
            jax.ShapeDtypeStruct((N, D), jnp.float32),
            jax.ShapeDtypeStruct((N, 1), jnp.float32),
        ],
    )(acc2, acc2, y, dis, bc, wl, bl)


def kernel(x, edge_index, W_conv, b_conv, W_lin, b_lin):
    ei_flat = edge_index.astype(jnp.int32).reshape(2 * E)

    deg8 = _deg_kernel(ei_flat).reshape(8, N)
    dis = _dis_kernel(deg8)

    y = _prescale(x, W_conv, dis)
    acc2 = _edge_kernel(y, ei_flat)

    h, out = _finish(acc2, y, dis,
                     b_conv.reshape(1, D), W_lin, b_lin.reshape(1, 1))
    return (out, h)


# speedup vs baseline: 47.2663x; 1.0407x over previous
import functools

import jax
import jax.numpy as jnp
from jax import lax
from jax.experimental import pallas as pl
from jax.experimental.pallas import tpu as pltpu
from jax.experimental.pallas import tpu_sc as plsc

N = 10000
E = 320000
D = 128

NC = 2
NS = 16
ROWS_PER_TILE = 624
ROWS_TAIL = N - NS * ROWS_PER_TILE
CHUNK = 128

DEG_EDGES_PER_WORKER = E // (NC * NS)
DEG_NFULL = DEG_EDGES_PER_WORKER // CHUNK
DEG_TAIL = DEG_EDGES_PER_WORKER - DEG_NFULL * CHUNK


def _deg_kernel(ei_flat):
    @functools.partial(
        pl.kernel,
        out_type=jax.ShapeDtypeStruct((8 * N,), jnp.float32),
        mesh=plsc.VectorSubcoreMesh(core_axis_name="c", subcore_axis_name="s"),
        scratch_types=[
            pltpu.VMEM_SHARED((N,), jnp.float32),
            pltpu.VMEM((ROWS_PER_TILE,), jnp.float32),
            pltpu.VMEM((3, CHUNK), jnp.int32),
            pltpu.VMEM((CHUNK,), jnp.float32),
            pltpu.VMEM((DEG_TAIL,), jnp.int32),
            pltpu.SemaphoreType.DMA((3,)),
        ],
    )
    def k(dst_hbm, out_hbm, acc_sh, zeros_v, idx_v, ones_v, tidx_v, isem):
        cid = lax.axis_index("c")
        sid = lax.axis_index("s")

        def fill_zeros(i, carry):
            zeros_v[pl.ds(i * 16, 16)] = jnp.zeros((16,), jnp.float32)
            return carry

        lax.fori_loop(0, ROWS_PER_TILE // 16, fill_zeros, 0)

        def fill_ones(i, carry):
            ones_v[pl.ds(i * 16, 16)] = jnp.ones((16,), jnp.float32)
            return carry

        lax.fori_loop(0, CHUNK // 16, fill_ones, 0)

        row0 = pl.multiple_of(sid * ROWS_PER_TILE, 8)
        pltpu.sync_copy(zeros_v, acc_sh.at[pl.ds(row0, ROWS_PER_TILE)])

        @pl.when(sid == NS - 1)
        def _():
            pltpu.sync_copy(zeros_v.at[pl.ds(0, ROWS_TAIL)],
                            acc_sh.at[pl.ds(NS * ROWS_PER_TILE, ROWS_TAIL)])

        plsc.subcore_barrier()

        base0 = E + (cid * NS + sid) * DEG_EDGES_PER_WORKER

        def issue(i, b):
            pltpu.async_copy(dst_hbm.at[pl.ds(base0 + i * CHUNK, CHUNK)],
                             idx_v.at[b], isem.at[b])

        def wait(i, b):
            pltpu.make_async_copy(dst_hbm.at[pl.ds(base0 + i * CHUNK, CHUNK)],
                                  idx_v.at[b], isem.at[b]).wait()

        def scatter(b):
            pltpu.sync_copy(ones_v, acc_sh.at[idx_v.at[b]], add=True)

        tslice = dst_hbm.at[pl.ds(base0 + DEG_NFULL * CHUNK, DEG_TAIL)]

        issue(0, 0)
        issue(1, 1)
        issue(2, 2)

        def body(t, carry):
            i = 3 * t
            for b in range(3):
                wait(i + b, b)
                scatter(b)
                issue(i + b + 3, b)
            return carry

        lax.fori_loop(0, DEG_NFULL // 3 - 1, body, 0)
        wait(DEG_NFULL - 3, 0)
        scatter(0)
        pltpu.async_copy(tslice, tidx_v, isem.at[0])
        wait(DEG_NFULL - 2, 1)
        scatter(1)
        wait(DEG_NFULL - 1, 2)
        scatter(2)
        pltpu.make_async_copy(tslice, tidx_v, isem.at[0]).wait()
        pltpu.sync_copy(ones_v.at[pl.ds(0, DEG_TAIL)], acc_sh.at[tidx_v],
                        add=True)
        plsc.subcore_barrier()
        obase = pl.multiple_of(cid * N + row0, 8)
        pltpu.sync_copy(acc_sh.at[pl.ds(row0, ROWS_PER_TILE)], zeros_v)
        pltpu.sync_copy(zeros_v, out_hbm.at[pl.ds(obase, ROWS_PER_TILE)])

        @pl.when(sid == NS - 1)
        def _():
            pltpu.sync_copy(acc_sh.at[pl.ds(NS * ROWS_PER_TILE, ROWS_TAIL)],
                            zeros_v.at[pl.ds(0, ROWS_TAIL)])
            pltpu.sync_copy(
                zeros_v.at[pl.ds(0, ROWS_TAIL)],
                out_hbm.at[pl.ds(pl.multiple_of(cid * N + NS * ROWS_PER_TILE, 8),
                                 ROWS_TAIL)])

    return k(ei_flat)


ECHUNK = 80
E_NCHUNK = DEG_EDGES_PER_WORKER // ECHUNK
NBUF = 4


def _edge_kernel(y, ei_flat):
    @functools.partial(
        pl.kernel,
        out_type=jax.ShapeDtypeStruct((NC * N, D), jnp.float32),
        mesh=plsc.VectorSubcoreMesh(core_axis_name="c", subcore_axis_name="s"),
        scratch_types=[
            pltpu.VMEM_SHARED((N, D), jnp.float32),
            pltpu.VMEM((NBUF, ECHUNK), jnp.int32),
            pltpu.VMEM((NBUF, ECHUNK), jnp.int32),
            pltpu.VMEM((NBUF, ECHUNK), jnp.int32),
            pltpu.VMEM((NBUF, ECHUNK, D), jnp.float32),
            pltpu.SemaphoreType.DMA((NBUF,)),
            pltpu.SemaphoreType.DMA((NBUF,)),
            pltpu.SemaphoreType.DMA((NBUF,)),
        ],
    )
    def k(y_hbm, ei_hbm, out_hbm, acc_sh, sidx_v, didx_v, sdidx_v,
          rows_v, isem, gsem, ssem):
        cid = lax.axis_index("c")
        sid = lax.axis_index("s")

        def fill_zeros(i, carry):
            for j in range(D // 16):
                rows_v[0, i, pl.ds(j * 16, 16)] = jnp.zeros((16,), jnp.float32)
            return carry

        lax.fori_loop(0, ECHUNK, fill_zeros, 0)
        row0 = pl.multiple_of(sid * ROWS_PER_TILE, 8)
        for r in range(7):
            pltpu.sync_copy(rows_v.at[0],
                            acc_sh.at[pl.ds(row0 + r * ECHUNK, ECHUNK)])
        pltpu.sync_copy(rows_v.at[0, pl.ds(0, ROWS_PER_TILE - 7 * ECHUNK)],
                        acc_sh.at[pl.ds(row0 + 7 * ECHUNK,
                                        ROWS_PER_TILE - 7 * ECHUNK)])

        @pl.when(sid == NS - 1)
        def _():
            pltpu.sync_copy(rows_v.at[0, pl.ds(0, ROWS_TAIL)],
                            acc_sh.at[pl.ds(NS * ROWS_PER_TILE, ROWS_TAIL)])

        plsc.subcore_barrier()

        base0 = (cid * NS + sid) * DEG_EDGES_PER_WORKER
        LAST = E_NCHUNK - 1

        def idx_issue(i, b):
            base = base0 + i * ECHUNK
            pltpu.async_copy(ei_hbm.at[pl.ds(base, ECHUNK)], sidx_v.at[b],
                             isem.at[b])
            pltpu.async_copy(ei_hbm.at[pl.ds(E + base, ECHUNK)], didx_v.at[b],
                             isem.at[b])

        def idx_wait(i, b):
            base = base0 + i * ECHUNK
            pltpu.make_async_copy(ei_hbm.at[pl.ds(base, ECHUNK)], sidx_v.at[b],
                                  isem.at[b]).wait()
            pltpu.make_async_copy(ei_hbm.at[pl.ds(E + base, ECHUNK)],
                                  didx_v.at[b], isem.at[b]).wait()

        def dcopy(b):
            for j in range(ECHUNK // 16):
                sdidx_v[b, pl.ds(j * 16, 16)] = didx_v[b, pl.ds(j * 16, 16)]

        def gather_issue(b):
            pltpu.async_copy(y_hbm.at[sidx_v.at[b]], rows_v.at[b], gsem.at[b])

        def gather_wait(b):
            pltpu.make_async_copy(y_hbm.at[sidx_v.at[b]], rows_v.at[b],
                                  gsem.at[b]).wait()

        def scatter_issue(b):
            pltpu.async_copy(rows_v.at[b], acc_sh.at[sdidx_v.at[b]], ssem.at[b],
                             add=True)

        def scatter_wait(b):
            pltpu.make_async_copy(rows_v.at[b], acc_sh.at[sdidx_v.at[b]],
                                  ssem.at[b]).wait()

        def prep(i, b):
            idx_wait(i, b)
            dcopy(b)
            gather_issue(b)

        for i in range(NBUF):
            idx_issue(i, i)
        prep(0, 0)
        prep(1, 1)
        gather_wait(0)
        idx_issue(NBUF, 0)
        scatter_issue(0)
        prep(2, 2)
        gather_wait(1)
        idx_issue(NBUF + 1, 1)
        scatter_issue(1)
        prep(3, 3)

        def body(t, carry):
            j0 = 4 * t + 2
            for u in range(4):
                j = j0 + u
                b = (2 + u) % NBUF
                scatter_wait((2 + u + 2) % NBUF)
                prep(j + 2, (2 + u + 2) % NBUF)
                gather_wait(b)
                idx_issue(j + 4, b)
                scatter_issue(b)
            return carry

        lax.fori_loop(0, 29, body, 0)

        for j in range(118, 125):
            scatter_wait((j - 2) % NBUF)
            if j + 2 <= LAST:
                prep(j + 2, (j + 2) % NBUF)
            gather_wait(j % NBUF)
            if j + 4 <= LAST:
                idx_issue(j + 4, j % NBUF)
            scatter_issue(j % NBUF)
        scatter_wait(123 % NBUF)
        scatter_wait(124 % NBUF)
        plsc.subcore_barrier()

        obase = pl.multiple_of(cid * N + row0, 8)

        def cp_size(r):
            return ECHUNK if r < 7 else ROWS_PER_TILE - 7 * ECHUNK

        def cp_in(r, b):
            pltpu.async_copy(acc_sh.at[pl.ds(row0 + r * ECHUNK, cp_size(r))],
                             rows_v.at[b, pl.ds(0, cp_size(r))], gsem.at[b])

        def cp_in_wait(r, b):
            pltpu.make_async_copy(
                acc_sh.at[pl.ds(row0 + r * ECHUNK, cp_size(r))],
                rows_v.at[b, pl.ds(0, cp_size(r))], gsem.at[b]).wait()

        for r in range(NBUF):
            cp_in(r, r)
        for r in range(8):
            b = r % NBUF
            cp_in_wait(r, b)
            pltpu.sync_copy(rows_v.at[b, pl.ds(0, cp_size(r))],
                            out_hbm.at[pl.ds(obase + r * ECHUNK, cp_size(r))])
            if r + NBUF < 8:
                cp_in(r + NBUF, b)

        @pl.when(sid == NS - 1)
        def _():
            tb = pl.multiple_of(NS * ROWS_PER_TILE, 8)
            pltpu.sync_copy(acc_sh.at[pl.ds(tb, ROWS_TAIL)],
                            rows_v.at[0, pl.ds(0, ROWS_TAIL)])
            pltpu.sync_copy(rows_v.at[0, pl.ds(0, ROWS_TAIL)],
                            out_hbm.at[pl.ds(pl.multiple_of(cid * N + tb, 8),
                                             ROWS_TAIL)])

    return k(y, ei_flat)


_BN = 1000


def _matmul_body(x_ref, w_ref, o_ref):
    o_ref[...] = jnp.dot(x_ref[...], w_ref[...],
                         preferred_element_type=jnp.float32)


def _matmul(x, w):
    return pl.pallas_call(
        _matmul_body,
        grid=(N // _BN,),
        in_specs=[
            pl.BlockSpec((_BN, D), lambda i: (i, 0)),
            pl.BlockSpec((D, D), lambda i: (0, 0)),
        ],
        out_specs=pl.BlockSpec((_BN, D), lambda i: (i, 0)),
        out_shape=jax.ShapeDtypeStruct((N, D), jnp.float32),
    )(x, w)


def _scale_body(xw_ref, d_ref, y_ref, dis_ref):
    dsum = d_ref[0:1, :] + d_ref[1:2, :] + 1.0
    dis = lax.transpose(lax.rsqrt(dsum), (1, 0))
    dis_ref[...] = dis
    y_ref[...] = xw_ref[...] * dis


def _scale(xw, deg8):
    return pl.pallas_call(
        _scale_body,
        grid=(1,),
        in_specs=[
            pl.BlockSpec((N, D), lambda i: (0, 0)),
            pl.BlockSpec((8, N), lambda i: (0, 0)),
        ],
        out_specs=[
            pl.BlockSpec((N, D), lambda i: (0, 0)),
            pl.BlockSpec((N, 1), lambda i: (0, 0)),
        ],
        out_shape=[
            jax.ShapeDtypeStruct((N, D), jnp.float32),
            jax.ShapeDtypeStruct((N, 1), jnp.float32),
        ],
    )(xw, deg8)


def _finish_body(a0_ref, a1_ref, y_ref, d_ref, bc_ref, wl_ref, bl_ref,
                 h_ref, o_ref):
    dis = d_ref[...]
    s = (a0_ref[...] + a1_ref[...] + y_ref[...]) * dis + bc_ref[...]
    h = jnp.maximum(s, 0.0)
    h_ref[...] = h
    z = jnp.dot(h, wl_ref[...], preferred_element_type=jnp.float32) + bl_ref[...]
    o_ref[...] = jax.nn.sigmoid(z)


def _finish(acc2, y, dis, bc, wl, bl):
    nb = N // _BN
    return pl.pallas_call(
        _finish_body,
        grid=(N // _BN,),
        in_specs=[
            pl.BlockSpec((_BN, D), lambda i: (i, 0)),
            pl.BlockSpec((_BN, D), lambda i: (nb + i, 0)),
            pl.BlockSpec((_BN, D), lambda i: (i, 0)),
            pl.BlockSpec((_BN, 1), lambda i: (i, 0)),
            pl.BlockSpec((1, D), lambda i: (0, 0)),
            pl.BlockSpec((D, 1), lambda i: (0, 0)),
            pl.BlockSpec((1, 1), lambda i: (0, 0)),
        ],
        out_specs=[
            pl.BlockSpec((_BN, D), lambda i: (i, 0)),
            pl.BlockSpec((_BN, 1), lambda i: (i, 0)),
        ],
        out_shape=[
            jax.ShapeDtypeStruct((N, D), jnp.float32),
            jax.ShapeDtypeStruct((N, 1), jnp.float32),
        ],
    )(acc2, acc2, y, dis, bc, wl, bl)


def kernel(x, edge_index, W_conv, b_conv, W_lin, b_lin):
    ei_flat = edge_index.astype(jnp.int32).reshape(2 * E)

    xw = _matmul(x, W_conv)
    deg8 = _deg_kernel(ei_flat).reshape(8, N)
    y, dis = _scale(xw, deg8)
    acc2 = _edge_kernel(y, ei_flat)

    h, out = _finish(acc2, y, dis,
                     b_conv.reshape(1, D), W_lin, b_lin.reshape(1, 1))
    return (out, h)


# speedup vs baseline: 47.4214x; 1.0033x over previous
import functools

import jax
import jax.numpy as jnp
from jax import lax
from jax.experimental import pallas as pl
from jax.experimental.pallas import tpu as pltpu
from jax.experimental.pallas import tpu_sc as plsc

N = 10000
E = 320000
D = 128

NC = 2
NS = 16
ROWS_PER_TILE = 624
ROWS_TAIL = N - NS * ROWS_PER_TILE
CHUNK = 128

DEG_EDGES_PER_WORKER = E // (NC * NS)
DEG_NFULL = DEG_EDGES_PER_WORKER // CHUNK
DEG_TAIL = DEG_EDGES_PER_WORKER - DEG_NFULL * CHUNK


def _deg_kernel(ei_flat):
    @functools.partial(
        pl.kernel,
        out_type=jax.ShapeDtypeStruct((8 * N,), jnp.float32),
        mesh=plsc.VectorSubcoreMesh(core_axis_name="c", subcore_axis_name="s"),
        scratch_types=[
            pltpu.VMEM_SHARED((N,), jnp.float32),
            pltpu.VMEM((ROWS_PER_TILE,), jnp.float32),
            pltpu.VMEM((3, CHUNK), jnp.int32),
            pltpu.VMEM((CHUNK,), jnp.float32),
            pltpu.VMEM((DEG_TAIL,), jnp.int32),
            pltpu.SemaphoreType.DMA((3,)),
        ],
    )
    def k(dst_hbm, out_hbm, acc_sh, zeros_v, idx_v, ones_v, tidx_v, isem):
        cid = lax.axis_index("c")
        sid = lax.axis_index("s")

        def fill_zeros(i, carry):
            zeros_v[pl.ds(i * 16, 16)] = jnp.zeros((16,), jnp.float32)
            return carry

        lax.fori_loop(0, ROWS_PER_TILE // 16, fill_zeros, 0)

        def fill_ones(i, carry):
            ones_v[pl.ds(i * 16, 16)] = jnp.ones((16,), jnp.float32)
            return carry

        lax.fori_loop(0, CHUNK // 16, fill_ones, 0)

        row0 = pl.multiple_of(sid * ROWS_PER_TILE, 8)
        pltpu.sync_copy(zeros_v, acc_sh.at[pl.ds(row0, ROWS_PER_TILE)])

        @pl.when(sid == NS - 1)
        def _():
            pltpu.sync_copy(zeros_v.at[pl.ds(0, ROWS_TAIL)],
                            acc_sh.at[pl.ds(NS * ROWS_PER_TILE, ROWS_TAIL)])

        plsc.subcore_barrier()

        base0 = E + (cid * NS + sid) * DEG_EDGES_PER_WORKER

        def issue(i, b):
            pltpu.async_copy(dst_hbm.at[pl.ds(base0 + i * CHUNK, CHUNK)],
                             idx_v.at[b], isem.at[b])

        def wait(i, b):
            pltpu.make_async_copy(dst_hbm.at[pl.ds(base0 + i * CHUNK, CHUNK)],
                                  idx_v.at[b], isem.at[b]).wait()

        def scatter(b):
            pltpu.sync_copy(ones_v, acc_sh.at[idx_v.at[b]], add=True)

        tslice = dst_hbm.at[pl.ds(base0 + DEG_NFULL * CHUNK, DEG_TAIL)]

        issue(0, 0)
        issue(1, 1)
        issue(2, 2)

        def body(t, carry):
            i = 3 * t
            for b in range(3):
                wait(i + b, b)
                scatter(b)
                issue(i + b + 3, b)
            return carry

        lax.fori_loop(0, DEG_NFULL // 3 - 1, body, 0)
        wait(DEG_NFULL - 3, 0)
        scatter(0)
        pltpu.async_copy(tslice, tidx_v, isem.at[0])
        wait(DEG_NFULL - 2, 1)
        scatter(1)
        wait(DEG_NFULL - 1, 2)
        scatter(2)
        pltpu.make_async_copy(tslice, tidx_v, isem.at[0]).wait()
        pltpu.sync_copy(ones_v.at[pl.ds(0, DEG_TAIL)], acc_sh.at[tidx_v],
                        add=True)
        plsc.subcore_barrier()
        obase = pl.multiple_of(cid * N + row0, 8)
        pltpu.sync_copy(acc_sh.at[pl.ds(row0, ROWS_PER_TILE)], zeros_v)
        pltpu.sync_copy(zeros_v, out_hbm.at[pl.ds(obase, ROWS_PER_TILE)])

        @pl.when(sid == NS - 1)
        def _():
            pltpu.sync_copy(acc_sh.at[pl.ds(NS * ROWS_PER_TILE, ROWS_TAIL)],
                            zeros_v.at[pl.ds(0, ROWS_TAIL)])
            pltpu.sync_copy(
                zeros_v.at[pl.ds(0, ROWS_TAIL)],
                out_hbm.at[pl.ds(pl.multiple_of(cid * N + NS * ROWS_PER_TILE, 8),
                                 ROWS_TAIL)])

    return k(ei_flat)


ECHUNK = 80
E_NCHUNK = DEG_EDGES_PER_WORKER // ECHUNK
NBUF = 4


def _edge_kernel(y, ei_flat):
    @functools.partial(
        pl.kernel,
        out_type=jax.ShapeDtypeStruct((NC * N, D), jnp.float32),
        mesh=plsc.VectorSubcoreMesh(core_axis_name="c", subcore_axis_name="s"),
        scratch_types=[
            pltpu.VMEM_SHARED((N, D), jnp.float32),
            pltpu.VMEM((NBUF, ECHUNK), jnp.int32),
            pltpu.VMEM((NBUF, ECHUNK), jnp.int32),
            pltpu.VMEM((NBUF, ECHUNK), jnp.int32),
            pltpu.VMEM((NBUF, ECHUNK, D), jnp.float32),
            pltpu.SemaphoreType.DMA((NBUF,)),
            pltpu.SemaphoreType.DMA((NBUF,)),
            pltpu.SemaphoreType.DMA((NBUF,)),
        ],
    )
    def k(y_hbm, ei_hbm, out_hbm, acc_sh, sidx_v, didx_v, sdidx_v,
          rows_v, isem, gsem, ssem):
        cid = lax.axis_index("c")
        sid = lax.axis_index("s")

        base0 = (cid * NS + sid) * DEG_EDGES_PER_WORKER
        LAST = E_NCHUNK - 1

        def idx_issue(i, b):
            base = base0 + i * ECHUNK
            pltpu.async_copy(ei_hbm.at[pl.ds(base, ECHUNK)], sidx_v.at[b],
                             isem.at[b])
            pltpu.async_copy(ei_hbm.at[pl.ds(E + base, ECHUNK)], didx_v.at[b],
                             isem.at[b])

        for i in range(NBUF):
            idx_issue(i, i)

        def fill_zeros(i, carry):
            for j in range(D // 16):
                rows_v[0, i, pl.ds(j * 16, 16)] = jnp.zeros((16,), jnp.float32)
            return carry

        lax.fori_loop(0, ECHUNK, fill_zeros, 0)
        row0 = pl.multiple_of(sid * ROWS_PER_TILE, 8)

        def zcp(r):
            sz = ECHUNK if r < 7 else ROWS_PER_TILE - 7 * ECHUNK
            return (rows_v.at[0, pl.ds(0, sz)],
                    acc_sh.at[pl.ds(row0 + r * ECHUNK, sz)], ssem.at[r % NBUF])

        for r in range(8):
            src, dst, sem = zcp(r)
            pltpu.async_copy(src, dst, sem)
        for r in range(8):
            src, dst, sem = zcp(r)
            pltpu.make_async_copy(src, dst, sem).wait()

        @pl.when(sid == NS - 1)
        def _():
            pltpu.sync_copy(rows_v.at[0, pl.ds(0, ROWS_TAIL)],
                            acc_sh.at[pl.ds(NS * ROWS_PER_TILE, ROWS_TAIL)])

        plsc.subcore_barrier()

        def idx_wait(i, b):
            base = base0 + i * ECHUNK
            pltpu.make_async_copy(ei_hbm.at[pl.ds(base, ECHUNK)], sidx_v.at[b],
                                  isem.at[b]).wait()
            pltpu.make_async_copy(ei_hbm.at[pl.ds(E + base, ECHUNK)],
                                  didx_v.at[b], isem.at[b]).wait()

        def dcopy(b):
            for j in range(ECHUNK // 16):
                sdidx_v[b, pl.ds(j * 16, 16)] = didx_v[b, pl.ds(j * 16, 16)]

        def gather_issue(b):
            pltpu.async_copy(y_hbm.at[sidx_v.at[b]], rows_v.at[b], gsem.at[b])

        def gather_wait(b):
            pltpu.make_async_copy(y_hbm.at[sidx_v.at[b]], rows_v.at[b],
                                  gsem.at[b]).wait()

        def scatter_issue(b):
            pltpu.async_copy(rows_v.at[b], acc_sh.at[sdidx_v.at[b]], ssem.at[b],
                             add=True)

        def scatter_wait(b):
            pltpu.make_async_copy(rows_v.at[b], acc_sh.at[sdidx_v.at[b]],
                                  ssem.at[b]).wait()

        def prep(i, b):
            idx_wait(i, b)
            dcopy(b)
            gather_issue(b)

        prep(0, 0)
        prep(1, 1)
        gather_wait(0)
        idx_issue(NBUF, 0)
        scatter_issue(0)
        prep(2, 2)
        gather_wait(1)
        idx_issue(NBUF + 1, 1)
        scatter_issue(1)
        prep(3, 3)

        def body(t, carry):
            j0 = 4 * t + 2
            for u in range(4):
                j = j0 + u
                b = (2 + u) % NBUF
                scatter_wait((2 + u + 2) % NBUF)
                prep(j + 2, (2 + u + 2) % NBUF)
                gather_wait(b)
                idx_issue(j + 4, b)
                scatter_issue(b)
            return carry

        lax.fori_loop(0, 29, body, 0)

        for j in range(118, 125):
            scatter_wait((j - 2) % NBUF)
            if j + 2 <= LAST:
                prep(j + 2, (j + 2) % NBUF)
            gather_wait(j % NBUF)
            if j + 4 <= LAST:
                idx_issue(j + 4, j % NBUF)
            scatter_issue(j % NBUF)
        scatter_wait(123 % NBUF)
        scatter_wait(124 % NBUF)
        plsc.subcore_barrier()

        obase = pl.multiple_of(cid * N + row0, 8)

        def cp_size(r):
            return ECHUNK if r < 7 else ROWS_PER_TILE - 7 * ECHUNK

        def cp_in(r, b):
            pltpu.async_copy(acc_sh.at[pl.ds(row0 + r * ECHUNK, cp_size(r))],
                             rows_v.at[b, pl.ds(0, cp_size(r))], gsem.at[b])

        def cp_in_wait(r, b):
            pltpu.make_async_copy(
                acc_sh.at[pl.ds(row0 + r * ECHUNK, cp_size(r))],
                rows_v.at[b, pl.ds(0, cp_size(r))], gsem.at[b]).wait()

        for r in range(NBUF):
            cp_in(r, r)
        for r in range(8):
            b = r % NBUF
            cp_in_wait(r, b)
            pltpu.sync_copy(rows_v.at[b, pl.ds(0, cp_size(r))],
                            out_hbm.at[pl.ds(obase + r * ECHUNK, cp_size(r))])
            if r + NBUF < 8:
                cp_in(r + NBUF, b)

        @pl.when(sid == NS - 1)
        def _():
            tb = pl.multiple_of(NS * ROWS_PER_TILE, 8)
            pltpu.sync_copy(acc_sh.at[pl.ds(tb, ROWS_TAIL)],
                            rows_v.at[0, pl.ds(0, ROWS_TAIL)])
            pltpu.sync_copy(rows_v.at[0, pl.ds(0, ROWS_TAIL)],
                            out_hbm.at[pl.ds(pl.multiple_of(cid * N + tb, 8),
                                             ROWS_TAIL)])

    return k(y, ei_flat)


_BN = 1000


def _matmul_body(x_ref, w_ref, o_ref):
    o_ref[...] = jnp.dot(x_ref[...], w_ref[...],
                         preferred_element_type=jnp.float32)


def _matmul(x, w):
    return pl.pallas_call(
        _matmul_body,
        grid=(N // _BN,),
        in_specs=[
            pl.BlockSpec((_BN, D), lambda i: (i, 0)),
            pl.BlockSpec((D, D), lambda i: (0, 0)),
        ],
        out_specs=pl.BlockSpec((_BN, D), lambda i: (i, 0)),
        out_shape=jax.ShapeDtypeStruct((N, D), jnp.float32),
    )(x, w)


def _scale_body(xw_ref, d_ref, y_ref, dis_ref):
    dsum = d_ref[0:1, :] + d_ref[1:2, :] + 1.0
    dis = lax.transpose(lax.rsqrt(dsum), (1, 0))
    dis_ref[...] = dis
    y_ref[...] = xw_ref[...] * dis


def _scale(xw, deg8):
    return pl.pallas_call(
        _scale_body,
        grid=(1,),
        in_specs=[
            pl.BlockSpec((N, D), lambda i: (0, 0)),
            pl.BlockSpec((8, N), lambda i: (0, 0)),
        ],
        out_specs=[
            pl.BlockSpec((N, D), lambda i: (0, 0)),
            pl.BlockSpec((N, 1), lambda i: (0, 0)),
        ],
        out_shape=[
            jax.ShapeDtypeStruct((N, D), jnp.float32),
            jax.ShapeDtypeStruct((N, 1), jnp.float32),
        ],
    )(xw, deg8)


def _finish_body(a0_ref, a1_ref, y_ref, d_ref, bc_ref, wl_ref, bl_ref,
                 h_ref, o_ref):
    dis = d_ref[...]
    s = (a0_ref[...] + a1_ref[...] + y_ref[...]) * dis + bc_ref[...]
    h = jnp.maximum(s, 0.0)
    h_ref[...] = h
    z = jnp.dot(h, wl_ref[...], preferred_element_type=jnp.float32) + bl_ref[...]
    o_ref[...] = jax.nn.sigmoid(z)


def _finish(acc2, y, dis, bc, wl, bl):
    nb = N // _BN
    return pl.pallas_call(
        _finish_body,
        grid=(N // _BN,),
        in_specs=[
            pl.BlockSpec((_BN, D), lambda i: (i, 0)),
            pl.BlockSpec((_BN, D), lambda i: (nb + i, 0)),
            pl.BlockSpec((_BN, D), lambda i: (i, 0)),
            pl.BlockSpec((_BN, 1), lambda i: (i, 0)),
            pl.BlockSpec((1, D), lambda i: (0, 0)),
            pl.BlockSpec((D, 1), lambda i: (0, 0)),
            pl.BlockSpec((1, 1), lambda i: (0, 0)),
        ],
        out_specs=[
            pl.BlockSpec((_BN, D), lambda i: (i, 0)),
            pl.BlockSpec((_BN, 1), lambda i: (i, 0)),
        ],
        out_shape=[
            jax.ShapeDtypeStruct((N, D), jnp.float32),
            jax.ShapeDtypeStruct((N, 1), jnp.float32),
        ],
    )(acc2, acc2, y, dis, bc, wl, bl)


def kernel(x, edge_index, W_conv, b_conv, W_lin, b_lin):
    ei_flat = edge_index.astype(jnp.int32).reshape(2 * E)

    xw = _matmul(x, W_conv)
    deg8 = _deg_kernel(ei_flat).reshape(8, N)
    y, dis = _scale(xw, deg8)
    acc2 = _edge_kernel(y, ei_flat)

    h, out = _finish(acc2, y, dis,
                     b_conv.reshape(1, D), W_lin, b_lin.reshape(1, 1))
    return (out, h)


# speedup vs baseline: 48.3211x; 1.0190x over previous
import functools

import jax
import jax.numpy as jnp
from jax import lax
from jax.experimental import pallas as pl
from jax.experimental.pallas import tpu as pltpu
from jax.experimental.pallas import tpu_sc as plsc

N = 10000
E = 320000
D = 128

NC = 2
NS = 16
ROWS_PER_TILE = 624
ROWS_TAIL = N - NS * ROWS_PER_TILE
CHUNK = 128

DEG_EDGES_PER_WORKER = E // (NC * NS)
DEG_NFULL = DEG_EDGES_PER_WORKER // CHUNK
DEG_TAIL = DEG_EDGES_PER_WORKER - DEG_NFULL * CHUNK


def _deg_kernel(ei_flat):
    @functools.partial(
        pl.kernel,
        out_type=jax.ShapeDtypeStruct((8 * N,), jnp.float32),
        mesh=plsc.VectorSubcoreMesh(core_axis_name="c", subcore_axis_name="s"),
        scratch_types=[
            pltpu.VMEM_SHARED((N,), jnp.float32),
            pltpu.VMEM((ROWS_PER_TILE,), jnp.float32),
            pltpu.VMEM((6, CHUNK), jnp.int32),
            pltpu.VMEM((CHUNK,), jnp.float32),
            pltpu.VMEM((DEG_TAIL,), jnp.int32),
            pltpu.SemaphoreType.DMA((7,)),
            pltpu.SemaphoreType.DMA((6,)),
        ],
    )
    def k(dst_hbm, out_hbm, acc_sh, zeros_v, idx_v, ones_v, tidx_v, isem,
          ssem):
        cid = lax.axis_index("c")
        sid = lax.axis_index("s")

        base0 = E + (cid * NS + sid) * DEG_EDGES_PER_WORKER

        def issue(i, b):
            pltpu.async_copy(dst_hbm.at[pl.ds(base0 + i * CHUNK, CHUNK)],
                             idx_v.at[b], isem.at[b])

        def wait(i, b):
            pltpu.make_async_copy(dst_hbm.at[pl.ds(base0 + i * CHUNK, CHUNK)],
                                  idx_v.at[b], isem.at[b]).wait()

        def scatter_issue(b):
            pltpu.async_copy(ones_v, acc_sh.at[idx_v.at[b]], ssem.at[b],
                             add=True)

        def scatter_wait(b):
            pltpu.make_async_copy(ones_v, acc_sh.at[idx_v.at[b]],
                                  ssem.at[b]).wait()

        tslice = dst_hbm.at[pl.ds(base0 + DEG_NFULL * CHUNK, DEG_TAIL)]

        issue(0, 0)
        issue(1, 1)
        issue(2, 2)
        pltpu.async_copy(tslice, tidx_v, isem.at[6])

        def fill_zeros(i, carry):
            zeros_v[pl.ds(i * 16, 16)] = jnp.zeros((16,), jnp.float32)
            return carry

        lax.fori_loop(0, ROWS_PER_TILE // 16, fill_zeros, 0)

        def fill_ones(i, carry):
            ones_v[pl.ds(i * 16, 16)] = jnp.ones((16,), jnp.float32)
            return carry

        lax.fori_loop(0, CHUNK // 16, fill_ones, 0)

        row0 = pl.multiple_of(sid * ROWS_PER_TILE, 8)
        pltpu.sync_copy(zeros_v, acc_sh.at[pl.ds(row0, ROWS_PER_TILE)])

        @pl.when(sid == NS - 1)
        def _():
            pltpu.sync_copy(zeros_v.at[pl.ds(0, ROWS_TAIL)],
                            acc_sh.at[pl.ds(NS * ROWS_PER_TILE, ROWS_TAIL)])

        plsc.subcore_barrier()

        def step(j, u, first, last):
            wait(j, u)
            scatter_issue(u)
            if not first:
                scatter_wait((u + 3) % 6)
            if not last:
                issue(j + 3, (u + 3) % 6)

        for u in range(6):
            step(u, u, u < 3, False)

        def body(t, carry):
            i = 6 * t
            for u in range(6):
                step(i + u, u, False, False)
            return carry

        lax.fori_loop(1, DEG_NFULL // 6 - 1, body, 0)

        for u in range(6):
            step(72 + u, u, False, u >= 3)
        for u in range(3, 6):
            scatter_wait(u)

        pltpu.make_async_copy(tslice, tidx_v, isem.at[6]).wait()
        pltpu.sync_copy(ones_v.at[pl.ds(0, DEG_TAIL)], acc_sh.at[tidx_v],
                        add=True)
        plsc.subcore_barrier()
        obase = pl.multiple_of(cid * N + row0, 8)
        pltpu.sync_copy(acc_sh.at[pl.ds(row0, ROWS_PER_TILE)], zeros_v)
        pltpu.sync_copy(zeros_v, out_hbm.at[pl.ds(obase, ROWS_PER_TILE)])

        @pl.when(sid == NS - 1)
        def _():
            pltpu.sync_copy(acc_sh.at[pl.ds(NS * ROWS_PER_TILE, ROWS_TAIL)],
                            zeros_v.at[pl.ds(0, ROWS_TAIL)])
            pltpu.sync_copy(
                zeros_v.at[pl.ds(0, ROWS_TAIL)],
                out_hbm.at[pl.ds(pl.multiple_of(cid * N + NS * ROWS_PER_TILE, 8),
                                 ROWS_TAIL)])

    return k(ei_flat)


ECHUNK = 80
E_NCHUNK = DEG_EDGES_PER_WORKER // ECHUNK
NBUF = 4


def _edge_kernel(y, ei_flat):
    @functools.partial(
        pl.kernel,
        out_type=jax.ShapeDtypeStruct((NC * N, D), jnp.float32),
        mesh=plsc.VectorSubcoreMesh(core_axis_name="c", subcore_axis_name="s"),
        scratch_types=[
            pltpu.VMEM_SHARED((N, D), jnp.float32),
            pltpu.VMEM((NBUF, ECHUNK), jnp.int32),
            pltpu.VMEM((NBUF, ECHUNK), jnp.int32),
            pltpu.VMEM((NBUF, ECHUNK), jnp.int32),
            pltpu.VMEM((NBUF, ECHUNK, D), jnp.float32),
            pltpu.SemaphoreType.DMA((NBUF,)),
            pltpu.SemaphoreType.DMA((NBUF,)),
            pltpu.SemaphoreType.DMA((NBUF,)),
        ],
    )
    def k(y_hbm, ei_hbm, out_hbm, acc_sh, sidx_v, didx_v, sdidx_v,
          rows_v, isem, gsem, ssem):
        cid = lax.axis_index("c")
        sid = lax.axis_index("s")

        base0 = (cid * NS + sid) * DEG_EDGES_PER_WORKER
        LAST = E_NCHUNK - 1

        def idx_issue(i, b):
            base = base0 + i * ECHUNK
            pltpu.async_copy(ei_hbm.at[pl.ds(base, ECHUNK)], sidx_v.at[b],
                             isem.at[b])
            pltpu.async_copy(ei_hbm.at[pl.ds(E + base, ECHUNK)], didx_v.at[b],
                             isem.at[b])

        for i in range(NBUF):
            idx_issue(i, i)

        def fill_zeros(i, carry):
            for j in range(D // 16):
                rows_v[0, i, pl.ds(j * 16, 16)] = jnp.zeros((16,), jnp.float32)
            return carry

        lax.fori_loop(0, ECHUNK, fill_zeros, 0)
        row0 = pl.multiple_of(sid * ROWS_PER_TILE, 8)

        def zcp(r):
            sz = ECHUNK if r < 7 else ROWS_PER_TILE - 7 * ECHUNK
            return (rows_v.at[0, pl.ds(0, sz)],
                    acc_sh.at[pl.ds(row0 + r * ECHUNK, sz)], ssem.at[r % NBUF])

        for r in range(8):
            src, dst, sem = zcp(r)
            pltpu.async_copy(src, dst, sem)
        for r in range(8):
            src, dst, sem = zcp(r)
            pltpu.make_async_copy(src, dst, sem).wait()

        @pl.when(sid == NS - 1)
        def _():
            pltpu.sync_copy(rows_v.at[0, pl.ds(0, ROWS_TAIL)],
                            acc_sh.at[pl.ds(NS * ROWS_PER_TILE, ROWS_TAIL)])

        plsc.subcore_barrier()

        def idx_wait(i, b):
            base = base0 + i * ECHUNK
            pltpu.make_async_copy(ei_hbm.at[pl.ds(base, ECHUNK)], sidx_v.at[b],
                                  isem.at[b]).wait()
            pltpu.make_async_copy(ei_hbm.at[pl.ds(E + base, ECHUNK)],
                                  didx_v.at[b], isem.at[b]).wait()

        def dcopy(b):
            for j in range(ECHUNK // 16):
                sdidx_v[b, pl.ds(j * 16, 16)] = didx_v[b, pl.ds(j * 16, 16)]

        def gather_issue(b):
            pltpu.async_copy(y_hbm.at[sidx_v.at[b]], rows_v.at[b], gsem.at[b])

        def gather_wait(b):
            pltpu.make_async_copy(y_hbm.at[sidx_v.at[b]], rows_v.at[b],
                                  gsem.at[b]).wait()

        def scatter_issue(b):
            pltpu.async_copy(rows_v.at[b], acc_sh.at[sdidx_v.at[b]], ssem.at[b],
                             add=True)

        def scatter_wait(b):
            pltpu.make_async_copy(rows_v.at[b], acc_sh.at[sdidx_v.at[b]],
                                  ssem.at[b]).wait()

        def prep(i, b):
            idx_wait(i, b)
            dcopy(b)
            gather_issue(b)

        prep(0, 0)
        prep(1, 1)
        gather_wait(0)
        idx_issue(NBUF, 0)
        scatter_issue(0)
        prep(2, 2)
        gather_wait(1)
        idx_issue(NBUF + 1, 1)
        scatter_issue(1)
        prep(3, 3)

        def body(t, carry):
            j0 = 4 * t + 2
            for u in range(4):
                j = j0 + u
                b = (2 + u) % NBUF
                scatter_wait((2 + u + 2) % NBUF)
                prep(j + 2, (2 + u + 2) % NBUF)
                gather_wait(b)
                idx_issue(j + 4, b)
                scatter_issue(b)
            return carry

        lax.fori_loop(0, 29, body, 0)

        for j in range(118, 125):
            scatter_wait((j - 2) % NBUF)
            if j + 2 <= LAST:
                prep(j + 2, (j + 2) % NBUF)
            gather_wait(j % NBUF)
            if j + 4 <= LAST:
                idx_issue(j + 4, j % NBUF)
            scatter_issue(j % NBUF)
        scatter_wait(123 % NBUF)
        scatter_wait(124 % NBUF)
        plsc.subcore_barrier()

        obase = pl.multiple_of(cid * N + row0, 8)

        def cp_size(r):
            return ECHUNK if r < 7 else ROWS_PER_TILE - 7 * ECHUNK

        def cp_in(r, b):
            pltpu.async_copy(acc_sh.at[pl.ds(row0 + r * ECHUNK, cp_size(r))],
                             rows_v.at[b, pl.ds(0, cp_size(r))], gsem.at[b])

        def cp_in_wait(r, b):
            pltpu.make_async_copy(
                acc_sh.at[pl.ds(row0 + r * ECHUNK, cp_size(r))],
                rows_v.at[b, pl.ds(0, cp_size(r))], gsem.at[b]).wait()

        for r in range(NBUF):
            cp_in(r, r)
        for r in range(8):
            b = r % NBUF
            cp_in_wait(r, b)
            pltpu.sync_copy(rows_v.at[b, pl.ds(0, cp_size(r))],
                            out_hbm.at[pl.ds(obase + r * ECHUNK, cp_size(r))])
            if r + NBUF < 8:
                cp_in(r + NBUF, b)

        @pl.when(sid == NS - 1)
        def _():
            tb = pl.multiple_of(NS * ROWS_PER_TILE, 8)
            pltpu.sync_copy(acc_sh.at[pl.ds(tb, ROWS_TAIL)],
                            rows_v.at[0, pl.ds(0, ROWS_TAIL)])
            pltpu.sync_copy(rows_v.at[0, pl.ds(0, ROWS_TAIL)],
                            out_hbm.at[pl.ds(pl.multiple_of(cid * N + tb, 8),
                                             ROWS_TAIL)])

    return k(y, ei_flat)


_BN = 1000


def _matmul_body(x_ref, w_ref, o_ref):
    o_ref[...] = jnp.dot(x_ref[...], w_ref[...],
                         preferred_element_type=jnp.float32)


def _matmul(x, w):
    return pl.pallas_call(
        _matmul_body,
        grid=(N // _BN,),
        in_specs=[
            pl.BlockSpec((_BN, D), lambda i: (i, 0)),
            pl.BlockSpec((D, D), lambda i: (0, 0)),
        ],
        out_specs=pl.BlockSpec((_BN, D), lambda i: (i, 0)),
        out_shape=jax.ShapeDtypeStruct((N, D), jnp.float32),
    )(x, w)


def _scale_body(xw_ref, d_ref, y_ref, dis_ref):
    dsum = d_ref[0:1, :] + d_ref[1:2, :] + 1.0
    dis = lax.transpose(lax.rsqrt(dsum), (1, 0))
    dis_ref[...] = dis
    y_ref[...] = xw_ref[...] * dis


def _scale(xw, deg8):
    return pl.pallas_call(
        _scale_body,
        grid=(1,),
        in_specs=[
            pl.BlockSpec((N, D), lambda i: (0, 0)),
            pl.BlockSpec((8, N), lambda i: (0, 0)),
        ],
        out_specs=[
            pl.BlockSpec((N, D), lambda i: (0, 0)),
            pl.BlockSpec((N, 1), lambda i: (0, 0)),
        ],
        out_shape=[
            jax.ShapeDtypeStruct((N, D), jnp.float32),
            jax.ShapeDtypeStruct((N, 1), jnp.float32),
        ],
    )(xw, deg8)


def _finish_body(a0_ref, a1_ref, y_ref, d_ref, bc_ref, wl_ref, bl_ref,
                 h_ref, o_ref):
    dis = d_ref[...]
    s = (a0_ref[...] + a1_ref[...] + y_ref[...]) * dis + bc_ref[...]
    h = jnp.maximum(s, 0.0)
    h_ref[...] = h
    z = jnp.dot(h, wl_ref[...], preferred_element_type=jnp.float32) + bl_ref[...]
    o_ref[...] = jax.nn.sigmoid(z)


def _finish(acc2, y, dis, bc, wl, bl):
    nb = N // _BN
    return pl.pallas_call(
        _finish_body,
        grid=(N // _BN,),
        in_specs=[
            pl.BlockSpec((_BN, D), lambda i: (i, 0)),
            pl.BlockSpec((_BN, D), lambda i: (nb + i, 0)),
            pl.BlockSpec((_BN, D), lambda i: (i, 0)),
            pl.BlockSpec((_BN, 1), lambda i: (i, 0)),
            pl.BlockSpec((1, D), lambda i: (0, 0)),
            pl.BlockSpec((D, 1), lambda i: (0, 0)),
            pl.BlockSpec((1, 1), lambda i: (0, 0)),
        ],
        out_specs=[
            pl.BlockSpec((_BN, D), lambda i: (i, 0)),
            pl.BlockSpec((_BN, 1), lambda i: (i, 0)),
        ],
        out_shape=[
            jax.ShapeDtypeStruct((N, D), jnp.float32),
            jax.ShapeDtypeStruct((N, 1), jnp.float32),
        ],
    )(acc2, acc2, y, dis, bc, wl, bl)


def kernel(x, edge_index, W_conv, b_conv, W_lin, b_lin):
    ei_flat = edge_index.astype(jnp.int32).reshape(2 * E)

    xw = _matmul(x, W_conv)
    deg8 = _deg_kernel(ei_flat).reshape(8, N)
    y, dis = _scale(xw, deg8)
    acc2 = _edge_kernel(y, ei_flat)

    h, out = _finish(acc2, y, dis,
                     b_conv.reshape(1, D), W_lin, b_lin.reshape(1, 1))
    return (out, h)
